# Initial kernel scaffold; baseline (speedup 1.0000x reference)
import jax
import jax.numpy as jnp
from jax.experimental import pallas as pl


def kernel(feature, edges, weight, params):
    raise NotImplementedError("write your pallas kernel here")


# speedup vs baseline: 42.0617x; 42.0617x over previous
import functools

import jax
import jax.numpy as jnp
from jax import lax
from jax.experimental import pallas as pl
from jax.experimental.pallas import tpu as pltpu
from jax.experimental.pallas import tpu_sc as plsc

B = 64
NODE = 714
EPG = 8192
PLEN = 5099
CLEN = 400
PDIM = 300
CDIM = 100
CH = 128

NP = 768
NPNP = NP * NP
NC, NS = 2, 16
NW = NC * NS
TPS = NPNP // NS
EPT = EPG // NS
ECH = EPT // 128
ROWS_W = (B * NP) // NW
RCH = ROWS_W // 128

PPAD = 5104
CPAD = 408
OPAD = 5120


def _leaky(x, slope):
    return jnp.where(x >= 0, x, slope * x)


def _tc_proj(poi_pad, cat_pad, Wp, Wc):
    def body(poi_ref, cat_ref, wp_ref, wc_ref, op_ref, oc_ref):
        op_ref[...] = jnp.dot(poi_ref[...], wp_ref[...],
                              preferred_element_type=jnp.float32)
        oc_ref[...] = jnp.dot(cat_ref[...], wc_ref[...],
                              preferred_element_type=jnp.float32)

    return pl.pallas_call(
        body,
        out_shape=(jax.ShapeDtypeStruct((PPAD, CH), jnp.float32),
                   jax.ShapeDtypeStruct((CPAD, CH), jnp.float32)),
    )(poi_pad, cat_pad, Wp, Wc)


def _sc_gather(poi_proj, cat_proj, pidx, cidx):
    mesh = plsc.VectorSubcoreMesh(core_axis_name="c", subcore_axis_name="s")

    @functools.partial(
        pl.kernel,
        out_type=(jax.ShapeDtypeStruct((B * NP, CH), jnp.float32),
                  jax.ShapeDtypeStruct((B * NP, CH), jnp.float32)),
        mesh=mesh,
        scratch_types=[
            pltpu.VMEM((RCH, 128), jnp.int32),
            pltpu.VMEM((RCH, 128), jnp.int32),
            pltpu.VMEM((128, CH), jnp.float32),
            pltpu.VMEM((128, CH), jnp.float32),
            pltpu.SemaphoreType.DMA,
            pltpu.SemaphoreType.DMA,
        ],
    )
    def k(poi_hbm, cat_hbm, pidx_hbm, cidx_hbm, xpoi_hbm, xcat_hbm,
          pib, cib, prow, crow, sem0, sem1):
        wid = lax.axis_index("c") * NS + lax.axis_index("s")
        pltpu.sync_copy(pidx_hbm.at[wid], pib)
        pltpu.sync_copy(cidx_hbm.at[wid], cib)
        for ch in range(RCH):
            base = wid * ROWS_W + ch * 128
            cp = pltpu.async_copy(poi_hbm.at[pib.at[ch]], prow, sem0)
            cc = pltpu.async_copy(cat_hbm.at[cib.at[ch]], crow, sem1)
            cp.wait()
            pltpu.sync_copy(prow, xpoi_hbm.at[pl.ds(base, 128)])
            cc.wait()
            pltpu.sync_copy(crow, xcat_hbm.at[pl.ds(base, 128)])

    return k(poi_proj, cat_proj, pidx, cidx)


def _sc_build_adj(gidx, ew):
    mesh = plsc.VectorSubcoreMesh(core_axis_name="c", subcore_axis_name="s")
    GPC = B // NC

    @functools.partial(
        pl.kernel,
        out_type=(jax.ShapeDtypeStruct((B, NPNP), jnp.float32),
                  jax.ShapeDtypeStruct((B, NPNP), jnp.float32)),
        mesh=mesh,
        scratch_types=[
            pltpu.VMEM((ECH, 128), jnp.int32),
            pltpu.VMEM((ECH, 128), jnp.float32),
            pltpu.VMEM((ECH, 128), jnp.float32),
            pltpu.VMEM((128,), jnp.float32),
            pltpu.VMEM((128,), jnp.float32),
            pltpu.VMEM((4096,), jnp.float32),
            pltpu.VMEM_SHARED((NPNP,), jnp.float32),
            pltpu.VMEM_SHARED((NPNP,), jnp.float32),
        ],
    )
    def k(gidx_hbm, ew_hbm, outc_hbm, outw_hbm,
          ib, wb, nwb, ones, nones, zb, accc, accw):
        c = lax.axis_index("c")
        s = lax.axis_index("s")

        def fill16(ref, val, n):
            def bodyf(i, _):
                ref[pl.ds(i * 16, 16)] = jnp.full((16,), val, jnp.float32)
                return ()
            lax.fori_loop(0, n, bodyf, ())
        fill16(ones, 1.0, 8)
        fill16(nones, -1.0, 8)
        fill16(zb, 0.0, 256)

        def bodyz(i, _):
            off = s * TPS + i * 4096
            pltpu.sync_copy(zb, accc.at[pl.ds(off, 4096)])
            pltpu.sync_copy(zb, accw.at[pl.ds(off, 4096)])
            return ()
        lax.fori_loop(0, TPS // 4096, bodyz, ())
        plsc.subcore_barrier()

        def per_graph(j, _):
            g = c * GPC + j
            pltpu.sync_copy(gidx_hbm.at[g, s], ib)
            pltpu.sync_copy(ew_hbm.at[g, s], wb)
            for i in range(ECH):
                for kk in range(8):
                    sl = pl.ds(kk * 16, 16)
                    nwb[i, sl] = jnp.zeros((16,), jnp.float32) - wb[i, sl]
            for i in range(ECH):
                pltpu.sync_copy(ones, accc.at[ib.at[i]], add=True)
                pltpu.sync_copy(wb.at[i], accw.at[ib.at[i]], add=True)
            plsc.subcore_barrier()
            sl = pl.ds(s * TPS, TPS)
            pltpu.sync_copy(accc.at[sl], outc_hbm.at[g, sl])
            pltpu.sync_copy(accw.at[sl], outw_hbm.at[g, sl])
            plsc.subcore_barrier()
            for i in range(ECH):
                pltpu.sync_copy(nones, accc.at[ib.at[i]], add=True)
                pltpu.sync_copy(nwb.at[i], accw.at[ib.at[i]], add=True)
            plsc.subcore_barrier()
            return ()
        lax.fori_loop(0, GPC, per_graph, ())

    return k(gidx, ew)


def _tc_gnn(xpoi, xcat, f3, cnt, wsum, W3p, bin2, Wg_all, AS, AD, BG,
            Woutp, bo):
    def body(xp_ref, xc_ref, f3_ref, cnt_ref, ws_ref, w3_ref, bin_ref,
             wg_ref, as_ref, ad_ref, bg_ref, wo_ref, bo_ref, out_ref):
        cntm = cnt_ref[0]
        wsm = ws_ref[0]
        xin = (xp_ref[0] + xc_ref[0]
               + jnp.dot(f3_ref[0], w3_ref[...],
                         preferred_element_type=jnp.float32))
        deg = jnp.sum(wsm, axis=1, keepdims=True) + 1.0
        dis = 1.0 / jnp.sqrt(deg)
        dis2 = dis * dis

        def gcn_apply(h):
            t = jnp.dot(wsm, h * dis, preferred_element_type=jnp.float32)
            return dis * t + dis2 * h

        f = _leaky(gcn_apply(xin) + bin_ref[0][None, :], 0.01)

        r = lax.broadcasted_iota(jnp.int32, (NP, NP), 0)
        cidx = lax.broadcasted_iota(jnp.int32, (NP, NP), 1)
        cpi = cntm + jnp.where(r == cidx, 1.0, 0.0)
        valid = cpi > 0

        def gat(x, i):
            h = jnp.dot(x, wg_ref[i], preferred_element_type=jnp.float32)
            asrc = jnp.sum(h * as_ref[i][None, :], axis=1)
            adst = jnp.sum(h * ad_ref[i][None, :], axis=1)
            e = adst[:, None] + asrc[None, :]
            e = _leaky(e, 0.2)
            m = jnp.max(jnp.where(valid, e, -3e38), axis=1, keepdims=True)
            p = cpi * jnp.exp(jnp.minimum(e - m, 0.0))
            den = jnp.sum(p, axis=1, keepdims=True)
            agg = jnp.dot(p, h, preferred_element_type=jnp.float32)
            return agg / den + bg_ref[i][None, :]

        for i in range(3):
            t = gat(f, i)
            y = _leaky(t, 0.01) + t
            y2 = gat(y, i)
            f = _leaky(y2, 0.01)

        h2 = jnp.dot(f, wo_ref[...], preferred_element_type=jnp.float32)
        f2 = _leaky(gcn_apply(h2) + bo_ref[0][None, :], 0.01)
        out_ref[0, 0] = f2[:, 0]

    grid = (B,)
    return pl.pallas_call(
        body,
        grid=grid,
        in_specs=[
            pl.BlockSpec((1, NP, CH), lambda g: (g, 0, 0)),
            pl.BlockSpec((1, NP, CH), lambda g: (g, 0, 0)),
            pl.BlockSpec((1, NP, 8), lambda g: (g, 0, 0)),
            pl.BlockSpec((1, NP, NP), lambda g: (g, 0, 0)),
            pl.BlockSpec((1, NP, NP), lambda g: (g, 0, 0)),
            pl.BlockSpec((8, CH), lambda g: (0, 0)),
            pl.BlockSpec((1, CH), lambda g: (0, 0)),
            pl.BlockSpec((3, CH, CH), lambda g: (0, 0, 0)),
            pl.BlockSpec((8, CH), lambda g: (0, 0)),
            pl.BlockSpec((8, CH), lambda g: (0, 0)),
            pl.BlockSpec((8, CH), lambda g: (0, 0)),
            pl.BlockSpec((CH, 8), lambda g: (0, 0)),
            pl.BlockSpec((1, 8), lambda g: (0, 0)),
        ],
        out_specs=pl.BlockSpec((1, 1, NP), lambda g: (g, 0, 0)),
        out_shape=jax.ShapeDtypeStruct((B, 1, NP), jnp.float32),
    )(xpoi, xcat, f3, cnt, wsum, W3p, bin2, Wg_all, AS, AD, BG, Woutp, bo)


def _tc_head(fgr, W1p, b1, W2p, b2p):
    def body(f_ref, w1_ref, b1_ref, w2_ref, b2_ref, out_ref):
        h = jnp.maximum(
            jnp.dot(f_ref[...], w1_ref[...],
                    preferred_element_type=jnp.float32) + b1_ref[...], 0.0)
        out_ref[...] = jnp.maximum(
            jnp.dot(h, w2_ref[...],
                    preferred_element_type=jnp.float32) + b2_ref[...], 0.0)

    return pl.pallas_call(
        body,
        out_shape=jax.ShapeDtypeStruct((B, OPAD), jnp.float32),
    )(fgr, W1p, b1, W2p, b2p)


def kernel(feature, edges, weight, params):
    f32 = jnp.float32

    poi_idx = feature[:, :, 0].astype(jnp.int32)
    cat_idx = feature[:, :, 1].astype(jnp.int32)

    pidx = jnp.full((B, NP), PLEN, jnp.int32).at[:, :NODE].set(poi_idx)
    cidx = jnp.full((B, NP), CLEN, jnp.int32).at[:, :NODE].set(cat_idx)
    pidx = pidx.reshape(NW, RCH, 128)
    cidx = cidx.reshape(NW, RCH, 128)

    src = edges[:, 0, :]
    dst = edges[:, 1, :]
    gidx = (dst * NP + src).reshape(B, NS, ECH, 128)
    ew = weight[:, :, 1].reshape(B, NS, ECH, 128).astype(f32)

    W_in = params['W_in']
    poi_pad = jnp.zeros((PPAD, 304), f32).at[:PLEN, :PDIM].set(
        params['poi_table'])
    Wp = jnp.zeros((304, CH), f32).at[:PDIM].set(W_in[:PDIM])
    cat_pad = jnp.zeros((CPAD, 112), f32).at[:CLEN, :CDIM].set(
        params['cat_table'])
    Wc = jnp.zeros((112, CH), f32).at[:CDIM].set(W_in[PDIM:PDIM + CDIM])

    f3 = jnp.zeros((B, NP, 8), f32).at[:, :NODE, 0:3].set(feature[:, :, 2:5])
    W3p = jnp.zeros((8, CH), f32).at[0:3].set(W_in[PDIM + CDIM:])
    bin2 = params['b_in'][None, :]

    Wg_all = jnp.stack([params['Wg%d' % i] for i in range(3)])
    AS = jnp.zeros((8, CH), f32).at[0:3].set(
        jnp.stack([params['as%d' % i] for i in range(3)]))
    AD = jnp.zeros((8, CH), f32).at[0:3].set(
        jnp.stack([params['ad%d' % i] for i in range(3)]))
    BG = jnp.zeros((8, CH), f32).at[0:3].set(
        jnp.stack([params['bg%d' % i] for i in range(3)]))

    Woutp = jnp.zeros((CH, 8), f32).at[:, 0].set(params['W_out'][:, 0])
    bo = jnp.broadcast_to(params['b_out'][0], (1, 8))

    W1p = jnp.zeros((NP, CH), f32).at[:NODE].set(params['W1'])
    b1 = params['b1'][None, :]
    W2p = jnp.zeros((CH, OPAD), f32).at[:, :PLEN].set(params['W2'])
    b2p = jnp.zeros((1, OPAD), f32).at[0, :PLEN].set(params['b2'])

    poi_proj, cat_proj = _tc_proj(poi_pad, cat_pad, Wp, Wc)
    xpoi, xcat = _sc_gather(poi_proj, cat_proj, pidx, cidx)
    xpoi = xpoi.reshape(B, NP, CH)
    xcat = xcat.reshape(B, NP, CH)
    cntf, wsumf = _sc_build_adj(gidx, ew)
    cnt = cntf.reshape(B, NP, NP)
    wsum = wsumf.reshape(B, NP, NP)

    fgr = _tc_gnn(xpoi, xcat, f3, cnt, wsum, W3p, bin2, Wg_all, AS, AD, BG,
                  Woutp, bo).reshape(B, NP)
    out = _tc_head(fgr, W1p, b1, W2p, b2p)
    return out[:, :PLEN]


# speedup vs baseline: 42.5605x; 1.0119x over previous
import functools

import jax
import jax.numpy as jnp
from jax import lax
from jax.experimental import pallas as pl
from jax.experimental.pallas import tpu as pltpu
from jax.experimental.pallas import tpu_sc as plsc

B = 64
NODE = 714
EPG = 8192
PLEN = 5099
CLEN = 400
PDIM = 300
CDIM = 100
CH = 128

NP = 768
NPNP = NP * NP
NC, NS = 2, 16
NW = NC * NS
TPS = NPNP // NS
EPT = EPG // NS
ECH = EPT // 128
ROWS_W = (B * NP) // NW
RCH = ROWS_W // 128

PPAD = 5104
CPAD = 408
OPAD = 5120


def _leaky(x, slope):
    return jnp.where(x >= 0, x, slope * x)


def _tc_proj(poi_pad, cat_pad, Wp, Wc):
    def body(poi_ref, cat_ref, wp_ref, wc_ref, op_ref, oc_ref):
        op_ref[...] = jnp.dot(poi_ref[...], wp_ref[...],
                              preferred_element_type=jnp.float32)
        oc_ref[...] = jnp.dot(cat_ref[...], wc_ref[...],
                              preferred_element_type=jnp.float32)

    return pl.pallas_call(
        body,
        out_shape=(jax.ShapeDtypeStruct((PPAD, CH), jnp.float32),
                   jax.ShapeDtypeStruct((CPAD, CH), jnp.float32)),
    )(poi_pad, cat_pad, Wp, Wc)


def _sc_gather(poi_proj, cat_proj, pidx, cidx):
    mesh = plsc.VectorSubcoreMesh(core_axis_name="c", subcore_axis_name="s")

    @functools.partial(
        pl.kernel,
        out_type=jax.ShapeDtypeStruct((B * NP, CH), jnp.float32),
        mesh=mesh,
        scratch_types=[
            pltpu.VMEM((RCH, 128), jnp.int32),
            pltpu.VMEM((RCH, 128), jnp.int32),
            pltpu.VMEM((128, CH), jnp.float32),
            pltpu.VMEM((128, CH), jnp.float32),
            pltpu.SemaphoreType.DMA,
            pltpu.SemaphoreType.DMA,
            pltpu.SemaphoreType.DMA,
        ],
    )
    def k(poi_hbm, cat_hbm, pidx_hbm, cidx_hbm, xin_hbm,
          pib, cib, prow, crow, sem0, sem1, semw):
        wid = lax.axis_index("c") * NS + lax.axis_index("s")
        pltpu.sync_copy(pidx_hbm.at[wid], pib)
        pltpu.sync_copy(cidx_hbm.at[wid], cib)

        def chunk(ch, _):
            base = wid * ROWS_W + ch * 128
            cp = pltpu.async_copy(poi_hbm.at[pib.at[ch]], prow, sem0)
            cc = pltpu.async_copy(cat_hbm.at[cib.at[ch]], crow, sem1)
            cp.wait()
            cc.wait()

            def row(r, _):
                for kk in range(8):
                    sl = pl.ds(kk * 16, 16)
                    prow[r, sl] = prow[r, sl] + crow[r, sl]
                return ()
            lax.fori_loop(0, 128, row, ())
            pltpu.async_copy(prow, xin_hbm.at[pl.ds(base, 128)], semw).wait()
            return ()
        lax.fori_loop(0, RCH, chunk, ())

    return k(poi_proj, cat_proj, pidx, cidx)


def _sc_build_adj(gidx, ew):
    mesh = plsc.VectorSubcoreMesh(core_axis_name="c", subcore_axis_name="s")
    GPC = B // NC

    @functools.partial(
        pl.kernel,
        out_type=(jax.ShapeDtypeStruct((B, NPNP), jnp.float32),
                  jax.ShapeDtypeStruct((B, NPNP), jnp.float32)),
        mesh=mesh,
        scratch_types=[
            pltpu.VMEM((ECH, 128), jnp.int32),
            pltpu.VMEM((ECH, 128), jnp.float32),
            pltpu.VMEM((128,), jnp.float32),
            pltpu.VMEM((TPS,), jnp.float32),
            pltpu.VMEM_SHARED((NPNP,), jnp.float32),
            pltpu.VMEM_SHARED((NPNP,), jnp.float32),
            pltpu.SemaphoreType.DMA,
            pltpu.SemaphoreType.DMA,
            pltpu.SemaphoreType.DMA,
        ],
    )
    def k(gidx_hbm, ew_hbm, outc_hbm, outw_hbm,
          ib, wb, ones, zb, accc, accw, sems, semd, seme):
        c = lax.axis_index("c")
        s = lax.axis_index("s")

        def fill16(ref, val, n):
            def bodyf(i, _):
                ref[pl.ds(i * 16, 16)] = jnp.full((16,), val, jnp.float32)
                return ()
            lax.fori_loop(0, n, bodyf, ())
        fill16(ones, 1.0, 8)
        fill16(zb, 0.0, TPS // 16)

        sl = pl.ds(s * TPS, TPS)
        pltpu.async_copy(zb, accc.at[sl], semd)
        pltpu.async_copy(zb, accw.at[sl], semd)
        cpe0 = pltpu.async_copy(gidx_hbm.at[c * GPC, s], ib, seme)
        cpe1 = pltpu.async_copy(ew_hbm.at[c * GPC, s], wb, seme)
        pltpu.make_async_copy(zb, accc.at[sl], semd).wait()
        pltpu.make_async_copy(zb, accw.at[sl], semd).wait()
        cpe0.wait()
        cpe1.wait()
        plsc.subcore_barrier()

        def per_graph(j, _):
            g = c * GPC + j
            for i in range(ECH):
                pltpu.async_copy(ones, accc.at[ib.at[i]], sems, add=True)
                pltpu.async_copy(wb.at[i], accw.at[ib.at[i]], sems, add=True)
            for i in range(ECH):
                pltpu.make_async_copy(ones, accc.at[ib.at[i]], sems).wait()
                pltpu.make_async_copy(wb.at[i], accw.at[ib.at[i]], sems).wait()
            plsc.subcore_barrier()
            dc = pltpu.async_copy(accc.at[sl], outc_hbm.at[g, sl], semd)
            dw = pltpu.async_copy(accw.at[sl], outw_hbm.at[g, sl], semd)
            dc.wait()
            dw.wait()

            @pl.when(j + 1 < GPC)
            def _():
                ce0 = pltpu.async_copy(gidx_hbm.at[g + 1, s], ib, seme)
                ce1 = pltpu.async_copy(ew_hbm.at[g + 1, s], wb, seme)
                zc = pltpu.async_copy(zb, accc.at[sl], semd)
                zw = pltpu.async_copy(zb, accw.at[sl], semd)
                zc.wait()
                zw.wait()
                ce0.wait()
                ce1.wait()
            plsc.subcore_barrier()
            return ()
        lax.fori_loop(0, GPC, per_graph, ())

    return k(gidx, ew)


def _tc_gnn(xine, f3, cnt, wsum, W3p, bin2, Wg_all, AS, AD, BG,
            Woutp, bo):
    def body(xp_ref, f3_ref, cnt_ref, ws_ref, w3_ref, bin_ref,
             wg_ref, as_ref, ad_ref, bg_ref, wo_ref, bo_ref, out_ref):
        cntm = cnt_ref[0]
        wsm = ws_ref[0]
        xin = (xp_ref[0]
               + jnp.dot(f3_ref[0], w3_ref[...],
                         preferred_element_type=jnp.float32))
        deg = jnp.sum(wsm, axis=1, keepdims=True) + 1.0
        dis = 1.0 / jnp.sqrt(deg)
        dis2 = dis * dis

        def gcn_apply(h):
            t = jnp.dot(wsm, h * dis, preferred_element_type=jnp.float32)
            return dis * t + dis2 * h

        f = _leaky(gcn_apply(xin) + bin_ref[0][None, :], 0.01)

        r = lax.broadcasted_iota(jnp.int32, (NP, NP), 0)
        cidx = lax.broadcasted_iota(jnp.int32, (NP, NP), 1)
        cpi = cntm + jnp.where(r == cidx, 1.0, 0.0)
        valid = cpi > 0

        def gat(x, i):
            h = jnp.dot(x, wg_ref[i], preferred_element_type=jnp.float32)
            asrc = jnp.sum(h * as_ref[i][None, :], axis=1)
            adst = jnp.sum(h * ad_ref[i][None, :], axis=1)
            e = adst[:, None] + asrc[None, :]
            e = _leaky(e, 0.2)
            m = jnp.max(jnp.where(valid, e, -3e38), axis=1, keepdims=True)
            p = cpi * jnp.exp(jnp.minimum(e - m, 0.0))
            den = jnp.sum(p, axis=1, keepdims=True)
            agg = jnp.dot(p, h, preferred_element_type=jnp.float32)
            return agg / den + bg_ref[i][None, :]

        for i in range(3):
            t = gat(f, i)
            y = _leaky(t, 0.01) + t
            y2 = gat(y, i)
            f = _leaky(y2, 0.01)

        h2 = jnp.dot(f, wo_ref[...], preferred_element_type=jnp.float32)
        f2 = _leaky(gcn_apply(h2) + bo_ref[0][None, :], 0.01)
        out_ref[0, 0] = f2[:, 0]

    grid = (B,)
    return pl.pallas_call(
        body,
        grid=grid,
        in_specs=[
            pl.BlockSpec((1, NP, CH), lambda g: (g, 0, 0)),
            pl.BlockSpec((1, NP, 8), lambda g: (g, 0, 0)),
            pl.BlockSpec((1, NP, NP), lambda g: (g, 0, 0)),
            pl.BlockSpec((1, NP, NP), lambda g: (g, 0, 0)),
            pl.BlockSpec((8, CH), lambda g: (0, 0)),
            pl.BlockSpec((1, CH), lambda g: (0, 0)),
            pl.BlockSpec((3, CH, CH), lambda g: (0, 0, 0)),
            pl.BlockSpec((8, CH), lambda g: (0, 0)),
            pl.BlockSpec((8, CH), lambda g: (0, 0)),
            pl.BlockSpec((8, CH), lambda g: (0, 0)),
            pl.BlockSpec((CH, 8), lambda g: (0, 0)),
            pl.BlockSpec((1, 8), lambda g: (0, 0)),
        ],
        out_specs=pl.BlockSpec((1, 1, NP), lambda g: (g, 0, 0)),
        out_shape=jax.ShapeDtypeStruct((B, 1, NP), jnp.float32),
    )(xine, f3, cnt, wsum, W3p, bin2, Wg_all, AS, AD, BG, Woutp, bo)


def _tc_head(fgr, W1p, b1, W2p, b2p):
    def body(f_ref, w1_ref, b1_ref, w2_ref, b2_ref, out_ref):
        h = jnp.maximum(
            jnp.dot(f_ref[...], w1_ref[...],
                    preferred_element_type=jnp.float32) + b1_ref[...], 0.0)
        out_ref[...] = jnp.maximum(
            jnp.dot(h, w2_ref[...],
                    preferred_element_type=jnp.float32) + b2_ref[...], 0.0)

    return pl.pallas_call(
        body,
        out_shape=jax.ShapeDtypeStruct((B, OPAD), jnp.float32),
    )(fgr, W1p, b1, W2p, b2p)


def kernel(feature, edges, weight, params):
    f32 = jnp.float32

    poi_idx = feature[:, :, 0].astype(jnp.int32)
    cat_idx = feature[:, :, 1].astype(jnp.int32)

    nar = jnp.arange(NP, dtype=jnp.int32)[None, :]
    ppad = PLEN + nar % (PPAD - PLEN)
    cpad = CLEN + nar % (CPAD - CLEN)
    pidx = jnp.broadcast_to(ppad, (B, NP)).at[:, :NODE].set(poi_idx)
    cidx = jnp.broadcast_to(cpad, (B, NP)).at[:, :NODE].set(cat_idx)
    pidx = pidx.reshape(NW, RCH, 128)
    cidx = cidx.reshape(NW, RCH, 128)

    src = edges[:, 0, :]
    dst = edges[:, 1, :]
    gidx = (dst * NP + src).reshape(B, NS, ECH, 128)
    ew = weight[:, :, 1].reshape(B, NS, ECH, 128).astype(f32)

    W_in = params['W_in']
    poi_pad = jnp.zeros((PPAD, 304), f32).at[:PLEN, :PDIM].set(
        params['poi_table'])
    Wp = jnp.zeros((304, CH), f32).at[:PDIM].set(W_in[:PDIM])
    cat_pad = jnp.zeros((CPAD, 112), f32).at[:CLEN, :CDIM].set(
        params['cat_table'])
    Wc = jnp.zeros((112, CH), f32).at[:CDIM].set(W_in[PDIM:PDIM + CDIM])

    f3 = jnp.zeros((B, NP, 8), f32).at[:, :NODE, 0:3].set(feature[:, :, 2:5])
    W3p = jnp.zeros((8, CH), f32).at[0:3].set(W_in[PDIM + CDIM:])
    bin2 = params['b_in'][None, :]

    Wg_all = jnp.stack([params['Wg%d' % i] for i in range(3)])
    AS = jnp.zeros((8, CH), f32).at[0:3].set(
        jnp.stack([params['as%d' % i] for i in range(3)]))
    AD = jnp.zeros((8, CH), f32).at[0:3].set(
        jnp.stack([params['ad%d' % i] for i in range(3)]))
    BG = jnp.zeros((8, CH), f32).at[0:3].set(
        jnp.stack([params['bg%d' % i] for i in range(3)]))

    Woutp = jnp.zeros((CH, 8), f32).at[:, 0].set(params['W_out'][:, 0])
    bo = jnp.broadcast_to(params['b_out'][0], (1, 8))

    W1p = jnp.zeros((NP, CH), f32).at[:NODE].set(params['W1'])
    b1 = params['b1'][None, :]
    W2p = jnp.zeros((CH, OPAD), f32).at[:, :PLEN].set(params['W2'])
    b2p = jnp.zeros((1, OPAD), f32).at[0, :PLEN].set(params['b2'])

    poi_proj, cat_proj = _tc_proj(poi_pad, cat_pad, Wp, Wc)
    xine = _sc_gather(poi_proj, cat_proj, pidx, cidx).reshape(B, NP, CH)
    cntf, wsumf = _sc_build_adj(gidx, ew)
    cnt = cntf.reshape(B, NP, NP)
    wsum = wsumf.reshape(B, NP, NP)

    fgr = _tc_gnn(xine, f3, cnt, wsum, W3p, bin2, Wg_all, AS, AD, BG,
                  Woutp, bo).reshape(B, NP)
    out = _tc_head(fgr, W1p, b1, W2p, b2p)
    return out[:, :PLEN]


# speedup vs baseline: 45.5629x; 1.0705x over previous
import functools

import jax
import jax.numpy as jnp
from jax import lax
from jax.experimental import pallas as pl
from jax.experimental.pallas import tpu as pltpu
from jax.experimental.pallas import tpu_sc as plsc

B = 64
NODE = 714
EPG = 8192
PLEN = 5099
CLEN = 400
PDIM = 300
CDIM = 100
CH = 128

NP = 768
NPNP = NP * NP
NC, NS = 2, 16
NW = NC * NS
TPS = NPNP // NS
RPT = NP // NS
EPT = EPG // NS
ECH = EPT // 128
ROWS_W = (B * NP) // NW
RCH = ROWS_W // 128

PPAD = 5104
CPAD = 408
OPAD = 5120


def _leaky(x, slope):
    return jnp.where(x >= 0, x, slope * x)


def _tc_proj(poi_pad, cat_pad, Wp, Wc):
    def body(poi_ref, cat_ref, wp_ref, wc_ref, op_ref, oc_ref):
        op_ref[...] = jnp.dot(poi_ref[...], wp_ref[...],
                              preferred_element_type=jnp.float32)
        oc_ref[...] = jnp.dot(cat_ref[...], wc_ref[...],
                              preferred_element_type=jnp.float32)

    return pl.pallas_call(
        body,
        out_shape=(jax.ShapeDtypeStruct((PPAD, CH), jnp.float32),
                   jax.ShapeDtypeStruct((CPAD, CH), jnp.float32)),
    )(poi_pad, cat_pad, Wp, Wc)


def _sc_gather(poi_proj, cat_proj, pidx, cidx):
    mesh = plsc.VectorSubcoreMesh(core_axis_name="c", subcore_axis_name="s")

    @functools.partial(
        pl.kernel,
        out_type=jax.ShapeDtypeStruct((B * NP, CH), jnp.float32),
        mesh=mesh,
        scratch_types=[
            pltpu.VMEM((RCH, 128), jnp.int32),
            pltpu.VMEM((RCH, 128), jnp.int32),
            pltpu.VMEM((128, CH), jnp.float32),
            pltpu.VMEM((128, CH), jnp.float32),
            pltpu.SemaphoreType.DMA,
            pltpu.SemaphoreType.DMA,
            pltpu.SemaphoreType.DMA,
        ],
    )
    def k(poi_hbm, cat_hbm, pidx_hbm, cidx_hbm, xin_hbm,
          pib, cib, prow, crow, sem0, sem1, semw):
        wid = lax.axis_index("c") * NS + lax.axis_index("s")
        pltpu.sync_copy(pidx_hbm.at[wid], pib)
        pltpu.sync_copy(cidx_hbm.at[wid], cib)

        def chunk(ch, _):
            base = wid * ROWS_W + ch * 128
            cp = pltpu.async_copy(poi_hbm.at[pib.at[ch]], prow, sem0)
            cc = pltpu.async_copy(cat_hbm.at[cib.at[ch]], crow, sem1)
            cp.wait()
            cc.wait()

            def row(r, _):
                for kk in range(8):
                    sl = pl.ds(kk * 16, 16)
                    prow[r, sl] = prow[r, sl] + crow[r, sl]
                return ()
            lax.fori_loop(0, 128, row, ())
            pltpu.async_copy(prow, xin_hbm.at[pl.ds(base, 128)], semw).wait()
            return ()
        lax.fori_loop(0, RCH, chunk, ())

    return k(poi_proj, cat_proj, pidx, cidx)


def _sc_build_adj(gidx, ew):
    mesh = plsc.VectorSubcoreMesh(core_axis_name="c", subcore_axis_name="s")
    GPC = B // NC

    @functools.partial(
        pl.kernel,
        out_type=(jax.ShapeDtypeStruct((B, NP, NP), jnp.float32),
                  jax.ShapeDtypeStruct((B, NP, NP), jnp.float32)),
        mesh=mesh,
        scratch_types=[
            pltpu.VMEM((ECH, 128), jnp.int32),
            pltpu.VMEM((ECH, 128), jnp.float32),
            pltpu.VMEM((128,), jnp.float32),
            pltpu.VMEM((TPS,), jnp.float32),
            pltpu.VMEM_SHARED((NPNP,), jnp.float32),
            pltpu.VMEM_SHARED((NPNP,), jnp.float32),
            pltpu.SemaphoreType.DMA,
            pltpu.SemaphoreType.DMA,
            pltpu.SemaphoreType.DMA,
        ],
    )
    def k(gidx_hbm, ew_hbm, outc_hbm, outw_hbm,
          ib, wb, ones, zb, accc, accw, sems, semd, seme):
        c = lax.axis_index("c")
        s = lax.axis_index("s")

        def fill16(ref, val, n):
            def bodyf(i, _):
                ref[pl.ds(i * 16, 16)] = jnp.full((16,), val, jnp.float32)
                return ()
            lax.fori_loop(0, n, bodyf, ())
        fill16(ones, 1.0, 8)
        fill16(zb, 0.0, TPS // 16)

        sl = pl.ds(s * TPS, TPS)
        pltpu.async_copy(zb, accc.at[sl], semd)
        pltpu.async_copy(zb, accw.at[sl], semd)
        cpe0 = pltpu.async_copy(gidx_hbm.at[c * GPC, s], ib, seme)
        cpe1 = pltpu.async_copy(ew_hbm.at[c * GPC, s], wb, seme)
        pltpu.make_async_copy(zb, accc.at[sl], semd).wait()
        pltpu.make_async_copy(zb, accw.at[sl], semd).wait()
        cpe0.wait()
        cpe1.wait()
        plsc.subcore_barrier()

        def per_graph(j, _):
            g = c * GPC + j
            for i in range(ECH):
                pltpu.async_copy(ones, accc.at[ib.at[i]], sems, add=True)
                pltpu.async_copy(wb.at[i], accw.at[ib.at[i]], sems, add=True)
            for i in range(ECH):
                pltpu.make_async_copy(ones, accc.at[ib.at[i]], sems).wait()
                pltpu.make_async_copy(wb.at[i], accw.at[ib.at[i]], sems).wait()
            plsc.subcore_barrier()
            def drain_row(r, _):
                rr = s * RPT + r
                pltpu.async_copy(accc.at[pl.ds(rr * NP, NP)],
                                 outc_hbm.at[g, rr, :], semd)
                pltpu.async_copy(accw.at[pl.ds(rr * NP, NP)],
                                 outw_hbm.at[g, rr, :], semd)
                return ()
            lax.fori_loop(0, RPT, drain_row, ())

            def wait_row(r, _):
                pltpu.make_async_copy(accc.at[pl.ds(0, NP)],
                                      outc_hbm.at[g, 0, :], semd).wait()
                pltpu.make_async_copy(accw.at[pl.ds(0, NP)],
                                      outw_hbm.at[g, 0, :], semd).wait()
                return ()
            lax.fori_loop(0, RPT, wait_row, ())

            @pl.when(j + 1 < GPC)
            def _():
                ce0 = pltpu.async_copy(gidx_hbm.at[g + 1, s], ib, seme)
                ce1 = pltpu.async_copy(ew_hbm.at[g + 1, s], wb, seme)
                zc = pltpu.async_copy(zb, accc.at[sl], semd)
                zw = pltpu.async_copy(zb, accw.at[sl], semd)
                zc.wait()
                zw.wait()
                ce0.wait()
                ce1.wait()
            plsc.subcore_barrier()
            return ()
        lax.fori_loop(0, GPC, per_graph, ())

    return k(gidx, ew)


def _tc_gnn(xine, f3, cnt, wsum, W3p, bin2, Wg_all, AS, AD, BG,
            Woutp, bo):
    def body(xp_ref, f3_ref, cnt_ref, ws_ref, w3_ref, bin_ref,
             wg_ref, as_ref, ad_ref, bg_ref, wo_ref, bo_ref, out_ref):
        cntm = cnt_ref[0]
        wsm = ws_ref[0]
        xin = (xp_ref[0]
               + jnp.dot(f3_ref[0], w3_ref[...],
                         preferred_element_type=jnp.float32))
        deg = jnp.sum(wsm, axis=1, keepdims=True) + 1.0
        dis = 1.0 / jnp.sqrt(deg)
        dis2 = dis * dis

        def gcn_apply(h):
            t = jnp.dot(wsm, h * dis, preferred_element_type=jnp.float32)
            return dis * t + dis2 * h

        f = _leaky(gcn_apply(xin) + bin_ref[0][None, :], 0.01)

        r = lax.broadcasted_iota(jnp.int32, (NP, NP), 0)
        cidx = lax.broadcasted_iota(jnp.int32, (NP, NP), 1)
        cpi = cntm + jnp.where(r == cidx, 1.0, 0.0)
        valid = cpi > 0

        def gat(x, i):
            h = jnp.dot(x, wg_ref[i], preferred_element_type=jnp.float32)
            asrc = jnp.sum(h * as_ref[i][None, :], axis=1)
            adst = jnp.sum(h * ad_ref[i][None, :], axis=1)
            e = adst[:, None] + asrc[None, :]
            e = _leaky(e, 0.2)
            m = jnp.max(jnp.where(valid, e, -3e38), axis=1, keepdims=True)
            p = cpi * jnp.exp(jnp.minimum(e - m, 0.0))
            den = jnp.sum(p, axis=1, keepdims=True)
            agg = jnp.dot(p, h, preferred_element_type=jnp.float32)
            return agg / den + bg_ref[i][None, :]

        for i in range(3):
            t = gat(f, i)
            y = _leaky(t, 0.01) + t
            y2 = gat(y, i)
            f = _leaky(y2, 0.01)

        h2 = jnp.dot(f, wo_ref[...], preferred_element_type=jnp.float32)
        f2 = _leaky(gcn_apply(h2) + bo_ref[0][None, :], 0.01)
        out_ref[0, 0] = f2[:, 0]

    grid = (B,)
    return pl.pallas_call(
        body,
        grid=grid,
        in_specs=[
            pl.BlockSpec((1, NP, CH), lambda g: (g, 0, 0)),
            pl.BlockSpec((1, NP, 8), lambda g: (g, 0, 0)),
            pl.BlockSpec((1, NP, NP), lambda g: (g, 0, 0)),
            pl.BlockSpec((1, NP, NP), lambda g: (g, 0, 0)),
            pl.BlockSpec((8, CH), lambda g: (0, 0)),
            pl.BlockSpec((1, CH), lambda g: (0, 0)),
            pl.BlockSpec((3, CH, CH), lambda g: (0, 0, 0)),
            pl.BlockSpec((8, CH), lambda g: (0, 0)),
            pl.BlockSpec((8, CH), lambda g: (0, 0)),
            pl.BlockSpec((8, CH), lambda g: (0, 0)),
            pl.BlockSpec((CH, 8), lambda g: (0, 0)),
            pl.BlockSpec((1, 8), lambda g: (0, 0)),
        ],
        out_specs=pl.BlockSpec((1, 1, NP), lambda g: (g, 0, 0)),
        out_shape=jax.ShapeDtypeStruct((B, 1, NP), jnp.float32),
    )(xine, f3, cnt, wsum, W3p, bin2, Wg_all, AS, AD, BG, Woutp, bo)


def _tc_head(fgr, W1p, b1, W2p, b2p):
    def body(f_ref, w1_ref, b1_ref, w2_ref, b2_ref, out_ref):
        h = jnp.maximum(
            jnp.dot(f_ref[...], w1_ref[...],
                    preferred_element_type=jnp.float32) + b1_ref[...], 0.0)
        out_ref[...] = jnp.maximum(
            jnp.dot(h, w2_ref[...],
                    preferred_element_type=jnp.float32) + b2_ref[...], 0.0)

    return pl.pallas_call(
        body,
        out_shape=jax.ShapeDtypeStruct((B, OPAD), jnp.float32),
    )(fgr, W1p, b1, W2p, b2p)


def kernel(feature, edges, weight, params):
    f32 = jnp.float32

    poi_idx = feature[:, :, 0].astype(jnp.int32)
    cat_idx = feature[:, :, 1].astype(jnp.int32)

    nar = jnp.arange(NP, dtype=jnp.int32)[None, :]
    ppad = PLEN + nar % (PPAD - PLEN)
    cpad = CLEN + nar % (CPAD - CLEN)
    pidx = jnp.broadcast_to(ppad, (B, NP)).at[:, :NODE].set(poi_idx)
    cidx = jnp.broadcast_to(cpad, (B, NP)).at[:, :NODE].set(cat_idx)
    pidx = pidx.reshape(NW, RCH, 128)
    cidx = cidx.reshape(NW, RCH, 128)

    src = edges[:, 0, :]
    dst = edges[:, 1, :]
    gidx = (dst * NP + src).reshape(B, NS, ECH, 128)
    ew = weight[:, :, 1].reshape(B, NS, ECH, 128).astype(f32)

    W_in = params['W_in']
    poi_pad = jnp.zeros((PPAD, 304), f32).at[:PLEN, :PDIM].set(
        params['poi_table'])
    Wp = jnp.zeros((304, CH), f32).at[:PDIM].set(W_in[:PDIM])
    cat_pad = jnp.zeros((CPAD, 112), f32).at[:CLEN, :CDIM].set(
        params['cat_table'])
    Wc = jnp.zeros((112, CH), f32).at[:CDIM].set(W_in[PDIM:PDIM + CDIM])

    f3 = jnp.zeros((B, NP, 8), f32).at[:, :NODE, 0:3].set(feature[:, :, 2:5])
    W3p = jnp.zeros((8, CH), f32).at[0:3].set(W_in[PDIM + CDIM:])
    bin2 = params['b_in'][None, :]

    Wg_all = jnp.stack([params['Wg%d' % i] for i in range(3)])
    AS = jnp.zeros((8, CH), f32).at[0:3].set(
        jnp.stack([params['as%d' % i] for i in range(3)]))
    AD = jnp.zeros((8, CH), f32).at[0:3].set(
        jnp.stack([params['ad%d' % i] for i in range(3)]))
    BG = jnp.zeros((8, CH), f32).at[0:3].set(
        jnp.stack([params['bg%d' % i] for i in range(3)]))

    Woutp = jnp.zeros((CH, 8), f32).at[:, 0].set(params['W_out'][:, 0])
    bo = jnp.broadcast_to(params['b_out'][0], (1, 8))

    W1p = jnp.zeros((NP, CH), f32).at[:NODE].set(params['W1'])
    b1 = params['b1'][None, :]
    W2p = jnp.zeros((CH, OPAD), f32).at[:, :PLEN].set(params['W2'])
    b2p = jnp.zeros((1, OPAD), f32).at[0, :PLEN].set(params['b2'])

    poi_proj, cat_proj = _tc_proj(poi_pad, cat_pad, Wp, Wc)
    xine = _sc_gather(poi_proj, cat_proj, pidx, cidx).reshape(B, NP, CH)
    cnt, wsum = _sc_build_adj(gidx, ew)

    fgr = _tc_gnn(xine, f3, cnt, wsum, W3p, bin2, Wg_all, AS, AD, BG,
                  Woutp, bo).reshape(B, NP)
    out = _tc_head(fgr, W1p, b1, W2p, b2p)
    return out[:, :PLEN]


# speedup vs baseline: 45.5773x; 1.0003x over previous
import functools

import jax
import jax.numpy as jnp
from jax import lax
from jax.experimental import pallas as pl
from jax.experimental.pallas import tpu as pltpu
from jax.experimental.pallas import tpu_sc as plsc

B = 64
NODE = 714
EPG = 8192
PLEN = 5099
CLEN = 400
PDIM = 300
CDIM = 100
CH = 128

NP = 768
NPNP = NP * NP
NC, NS = 2, 16
NW = NC * NS
TPS = NPNP // NS
RPT = NP // NS
EPT = EPG // NS
ECH = EPT // 128
ROWS_W = (B * NP) // NW
RCH = ROWS_W // 128

PPAD = 5104
CPAD = 408
OPAD = 5120


def _leaky(x, slope):
    return jnp.where(x >= 0, x, slope * x)


def _tc_proj(poi_pad, cat_pad, Wp, Wc):
    def body(poi_ref, cat_ref, wp_ref, wc_ref, op_ref, oc_ref):
        op_ref[...] = jnp.dot(poi_ref[...], wp_ref[...],
                              preferred_element_type=jnp.float32)
        oc_ref[...] = jnp.dot(cat_ref[...], wc_ref[...],
                              preferred_element_type=jnp.float32)

    return pl.pallas_call(
        body,
        out_shape=(jax.ShapeDtypeStruct((PPAD, CH), jnp.float32),
                   jax.ShapeDtypeStruct((CPAD, CH), jnp.float32)),
    )(poi_pad, cat_pad, Wp, Wc)


def _sc_gather(poi_proj, cat_proj, pidx, cidx):
    mesh = plsc.VectorSubcoreMesh(core_axis_name="c", subcore_axis_name="s")

    @functools.partial(
        pl.kernel,
        out_type=jax.ShapeDtypeStruct((B * NP, CH), jnp.float32),
        mesh=mesh,
        scratch_types=[
            pltpu.VMEM((RCH, 128), jnp.int32),
            pltpu.VMEM((RCH, 128), jnp.int32),
            pltpu.VMEM((128, CH), jnp.float32),
            pltpu.VMEM((128, CH), jnp.float32),
            pltpu.SemaphoreType.DMA,
            pltpu.SemaphoreType.DMA,
            pltpu.SemaphoreType.DMA,
        ],
    )
    def k(poi_hbm, cat_hbm, pidx_hbm, cidx_hbm, xin_hbm,
          pib, cib, prow, crow, sem0, sem1, semw):
        wid = lax.axis_index("c") * NS + lax.axis_index("s")
        pltpu.sync_copy(pidx_hbm.at[wid], pib)
        pltpu.sync_copy(cidx_hbm.at[wid], cib)

        def chunk(ch, _):
            base = wid * ROWS_W + ch * 128
            cp = pltpu.async_copy(poi_hbm.at[pib.at[ch]], prow, sem0)
            cc = pltpu.async_copy(cat_hbm.at[cib.at[ch]], crow, sem1)
            cp.wait()
            cc.wait()

            def row(r, _):
                for kk in range(8):
                    sl = pl.ds(kk * 16, 16)
                    prow[r, sl] = prow[r, sl] + crow[r, sl]
                return ()
            lax.fori_loop(0, 128, row, ())
            pltpu.async_copy(prow, xin_hbm.at[pl.ds(base, 128)], semw).wait()
            return ()
        lax.fori_loop(0, RCH, chunk, ())

    return k(poi_proj, cat_proj, pidx, cidx)


def _sc_build_adj(gidx, ew):
    mesh = plsc.VectorSubcoreMesh(core_axis_name="c", subcore_axis_name="s")
    GPC = B // NC

    @functools.partial(
        pl.kernel,
        out_type=(jax.ShapeDtypeStruct((B, NP, NP), jnp.float32),
                  jax.ShapeDtypeStruct((B, NP, NP), jnp.float32)),
        mesh=mesh,
        scratch_types=[
            pltpu.VMEM((ECH, 128), jnp.int32),
            pltpu.VMEM((ECH, 128), jnp.float32),
            pltpu.VMEM((128,), jnp.float32),
            pltpu.VMEM((TPS,), jnp.float32),
            pltpu.VMEM_SHARED((NPNP,), jnp.float32),
            pltpu.VMEM_SHARED((NPNP,), jnp.float32),
            pltpu.SemaphoreType.DMA,
            pltpu.SemaphoreType.DMA,
            pltpu.SemaphoreType.DMA,
        ],
    )
    def k(gidx_hbm, ew_hbm, outc_hbm, outw_hbm,
          ib, wb, ones, zb, accc, accw, sems, semd, seme):
        c = lax.axis_index("c")
        s = lax.axis_index("s")

        def fill16(ref, val, n):
            def bodyf(i, _):
                ref[pl.ds(i * 16, 16)] = jnp.full((16,), val, jnp.float32)
                return ()
            lax.fori_loop(0, n, bodyf, ())
        fill16(ones, 1.0, 8)
        fill16(zb, 0.0, TPS // 16)

        sl = pl.ds(s * TPS, TPS)
        pltpu.async_copy(zb, accc.at[sl], semd)
        pltpu.async_copy(zb, accw.at[sl], semd)
        cpe0 = pltpu.async_copy(gidx_hbm.at[c * GPC, s], ib, seme)
        cpe1 = pltpu.async_copy(ew_hbm.at[c * GPC, s], wb, seme)
        pltpu.make_async_copy(zb, accc.at[sl], semd).wait()
        pltpu.make_async_copy(zb, accw.at[sl], semd).wait()
        cpe0.wait()
        cpe1.wait()
        plsc.subcore_barrier()

        def per_graph(j, _):
            g = c * GPC + j
            with jax.named_scope("adj_scatter"):
                for i in range(ECH):
                    pltpu.async_copy(ones, accc.at[ib.at[i]], sems, add=True)
                    pltpu.async_copy(wb.at[i], accw.at[ib.at[i]], sems,
                                     add=True)
                for i in range(ECH):
                    pltpu.make_async_copy(ones, accc.at[ib.at[i]],
                                          sems).wait()
                    pltpu.make_async_copy(wb.at[i], accw.at[ib.at[i]],
                                          sems).wait()
            plsc.subcore_barrier()
            with jax.named_scope("adj_drain"):
                def drain_row(r, _):
                    rr = s * RPT + r
                    pltpu.async_copy(accc.at[pl.ds(rr * NP, NP)],
                                     outc_hbm.at[g, rr, :], semd)
                    pltpu.async_copy(accw.at[pl.ds(rr * NP, NP)],
                                     outw_hbm.at[g, rr, :], semd)
                    return ()
                lax.fori_loop(0, RPT, drain_row, ())

                def wait_row(r, _):
                    pltpu.make_async_copy(accc.at[pl.ds(0, NP)],
                                          outc_hbm.at[g, 0, :], semd).wait()
                    pltpu.make_async_copy(accw.at[pl.ds(0, NP)],
                                          outw_hbm.at[g, 0, :], semd).wait()
                    return ()
                lax.fori_loop(0, RPT, wait_row, ())

            @pl.when(j + 1 < GPC)
            def _():
                with jax.named_scope("adj_refill"):
                    ce0 = pltpu.async_copy(gidx_hbm.at[g + 1, s], ib, seme)
                    ce1 = pltpu.async_copy(ew_hbm.at[g + 1, s], wb, seme)
                    zc = pltpu.async_copy(zb, accc.at[sl], semd)
                    zw = pltpu.async_copy(zb, accw.at[sl], semd)
                    zc.wait()
                    zw.wait()
                    ce0.wait()
                    ce1.wait()
            plsc.subcore_barrier()
            return ()
        lax.fori_loop(0, GPC, per_graph, ())

    return k(gidx, ew)


def _tc_gnn(xine, f3, cnt, wsum, W3p, bin2, Wg_all, AS, AD, BG,
            Woutp, bo):
    def body(xp_ref, f3_ref, cnt_ref, ws_ref, w3_ref, bin_ref,
             wg_ref, as_ref, ad_ref, bg_ref, wo_ref, bo_ref, out_ref):
        cntm = cnt_ref[0]
        wsm = ws_ref[0]
        xin = (xp_ref[0]
               + jnp.dot(f3_ref[0], w3_ref[...],
                         preferred_element_type=jnp.float32))
        deg = jnp.sum(wsm, axis=1, keepdims=True) + 1.0
        dis = 1.0 / jnp.sqrt(deg)
        dis2 = dis * dis

        def gcn_apply(h):
            t = jnp.dot(wsm, h * dis, preferred_element_type=jnp.float32)
            return dis * t + dis2 * h

        f = _leaky(gcn_apply(xin) + bin_ref[0][None, :], 0.01)

        r = lax.broadcasted_iota(jnp.int32, (NP, NP), 0)
        cidx = lax.broadcasted_iota(jnp.int32, (NP, NP), 1)
        cpi = cntm + jnp.where(r == cidx, 1.0, 0.0)
        valid = cpi > 0

        def gat(x, i):
            h = jnp.dot(x, wg_ref[i], preferred_element_type=jnp.float32)
            asrc = jnp.sum(h * as_ref[i][None, :], axis=1)
            adst = jnp.sum(h * ad_ref[i][None, :], axis=1)
            e = adst[:, None] + asrc[None, :]
            e = _leaky(e, 0.2)
            m = jnp.max(jnp.where(valid, e, -3e38), axis=1, keepdims=True)
            p = cpi * jnp.exp(jnp.minimum(e - m, 0.0))
            den = jnp.sum(p, axis=1, keepdims=True)
            agg = jnp.dot(p, h, preferred_element_type=jnp.float32)
            return agg / den + bg_ref[i][None, :]

        for i in range(3):
            t = gat(f, i)
            y = _leaky(t, 0.01) + t
            y2 = gat(y, i)
            f = _leaky(y2, 0.01)

        h2 = jnp.dot(f, wo_ref[...], preferred_element_type=jnp.float32)
        f2 = _leaky(gcn_apply(h2) + bo_ref[0][None, :], 0.01)
        out_ref[0, 0] = f2[:, 0]

    grid = (B,)
    return pl.pallas_call(
        body,
        grid=grid,
        in_specs=[
            pl.BlockSpec((1, NP, CH), lambda g: (g, 0, 0)),
            pl.BlockSpec((1, NP, 8), lambda g: (g, 0, 0)),
            pl.BlockSpec((1, NP, NP), lambda g: (g, 0, 0)),
            pl.BlockSpec((1, NP, NP), lambda g: (g, 0, 0)),
            pl.BlockSpec((8, CH), lambda g: (0, 0)),
            pl.BlockSpec((1, CH), lambda g: (0, 0)),
            pl.BlockSpec((3, CH, CH), lambda g: (0, 0, 0)),
            pl.BlockSpec((8, CH), lambda g: (0, 0)),
            pl.BlockSpec((8, CH), lambda g: (0, 0)),
            pl.BlockSpec((8, CH), lambda g: (0, 0)),
            pl.BlockSpec((CH, 8), lambda g: (0, 0)),
            pl.BlockSpec((1, 8), lambda g: (0, 0)),
        ],
        out_specs=pl.BlockSpec((1, 1, NP), lambda g: (g, 0, 0)),
        out_shape=jax.ShapeDtypeStruct((B, 1, NP), jnp.float32),
    )(xine, f3, cnt, wsum, W3p, bin2, Wg_all, AS, AD, BG, Woutp, bo)


def _tc_head(fgr, W1p, b1, W2p, b2p):
    def body(f_ref, w1_ref, b1_ref, w2_ref, b2_ref, out_ref):
        h = jnp.maximum(
            jnp.dot(f_ref[...], w1_ref[...],
                    preferred_element_type=jnp.float32) + b1_ref[...], 0.0)
        out_ref[...] = jnp.maximum(
            jnp.dot(h, w2_ref[...],
                    preferred_element_type=jnp.float32) + b2_ref[...], 0.0)

    return pl.pallas_call(
        body,
        out_shape=jax.ShapeDtypeStruct((B, OPAD), jnp.float32),
    )(fgr, W1p, b1, W2p, b2p)


def kernel(feature, edges, weight, params):
    f32 = jnp.float32

    poi_idx = feature[:, :, 0].astype(jnp.int32)
    cat_idx = feature[:, :, 1].astype(jnp.int32)

    nar = jnp.arange(NP, dtype=jnp.int32)[None, :]
    ppad = PLEN + nar % (PPAD - PLEN)
    cpad = CLEN + nar % (CPAD - CLEN)
    pidx = jnp.broadcast_to(ppad, (B, NP)).at[:, :NODE].set(poi_idx)
    cidx = jnp.broadcast_to(cpad, (B, NP)).at[:, :NODE].set(cat_idx)
    pidx = pidx.reshape(NW, RCH, 128)
    cidx = cidx.reshape(NW, RCH, 128)

    src = edges[:, 0, :]
    dst = edges[:, 1, :]
    gidx = (dst * NP + src).reshape(B, NS, ECH, 128)
    ew = weight[:, :, 1].reshape(B, NS, ECH, 128).astype(f32)

    W_in = params['W_in']
    poi_pad = jnp.zeros((PPAD, 304), f32).at[:PLEN, :PDIM].set(
        params['poi_table'])
    Wp = jnp.zeros((304, CH), f32).at[:PDIM].set(W_in[:PDIM])
    cat_pad = jnp.zeros((CPAD, 112), f32).at[:CLEN, :CDIM].set(
        params['cat_table'])
    Wc = jnp.zeros((112, CH), f32).at[:CDIM].set(W_in[PDIM:PDIM + CDIM])

    f3 = jnp.zeros((B, NP, 8), f32).at[:, :NODE, 0:3].set(feature[:, :, 2:5])
    W3p = jnp.zeros((8, CH), f32).at[0:3].set(W_in[PDIM + CDIM:])
    bin2 = params['b_in'][None, :]

    Wg_all = jnp.stack([params['Wg%d' % i] for i in range(3)])
    AS = jnp.zeros((8, CH), f32).at[0:3].set(
        jnp.stack([params['as%d' % i] for i in range(3)]))
    AD = jnp.zeros((8, CH), f32).at[0:3].set(
        jnp.stack([params['ad%d' % i] for i in range(3)]))
    BG = jnp.zeros((8, CH), f32).at[0:3].set(
        jnp.stack([params['bg%d' % i] for i in range(3)]))

    Woutp = jnp.zeros((CH, 8), f32).at[:, 0].set(params['W_out'][:, 0])
    bo = jnp.broadcast_to(params['b_out'][0], (1, 8))

    W1p = jnp.zeros((NP, CH), f32).at[:NODE].set(params['W1'])
    b1 = params['b1'][None, :]
    W2p = jnp.zeros((CH, OPAD), f32).at[:, :PLEN].set(params['W2'])
    b2p = jnp.zeros((1, OPAD), f32).at[0, :PLEN].set(params['b2'])

    poi_proj, cat_proj = _tc_proj(poi_pad, cat_pad, Wp, Wc)
    xine = _sc_gather(poi_proj, cat_proj, pidx, cidx).reshape(B, NP, CH)
    cnt, wsum = _sc_build_adj(gidx, ew)

    fgr = _tc_gnn(xine, f3, cnt, wsum, W3p, bin2, Wg_all, AS, AD, BG,
                  Woutp, bo).reshape(B, NP)
    out = _tc_head(fgr, W1p, b1, W2p, b2p)
    return out[:, :PLEN]


# speedup vs baseline: 97.2770x; 2.1343x over previous
import functools

import jax
import jax.numpy as jnp
from jax import lax
from jax.experimental import pallas as pl
from jax.experimental.pallas import tpu as pltpu
from jax.experimental.pallas import tpu_sc as plsc

B = 64
NODE = 714
EPG = 8192
PLEN = 5099
CLEN = 400
PDIM = 300
CDIM = 100
CH = 128

NP = 768
NPNP = NP * NP
NC, NS = 2, 16
NW = NC * NS
TPS = NPNP // NS
RPT = NP // NS
EPT = EPG // NS
ECH = EPT // 128
ROWS_W = (B * NP) // NW
RCH = ROWS_W // 128

PPAD = 5104
CPAD = 408
OPAD = 5120


def _leaky(x, slope):
    return jnp.where(x >= 0, x, slope * x)


def _tc_proj(poi_pad, cat_pad, Wp, Wc):
    def body(poi_ref, cat_ref, wp_ref, wc_ref, op_ref, oc_ref):
        op_ref[0] = jnp.dot(poi_ref[...], wp_ref[...],
                            preferred_element_type=jnp.float32)
        oc_ref[0] = jnp.dot(cat_ref[...], wc_ref[...],
                            preferred_element_type=jnp.float32)

    return pl.pallas_call(
        body,
        grid=(NW,),
        in_specs=[
            pl.BlockSpec((PPAD, 304), lambda w: (0, 0)),
            pl.BlockSpec((CPAD, 112), lambda w: (0, 0)),
            pl.BlockSpec((304, CH), lambda w: (0, 0)),
            pl.BlockSpec((112, CH), lambda w: (0, 0)),
        ],
        out_specs=(pl.BlockSpec((1, PPAD, CH), lambda w: (w, 0, 0)),
                   pl.BlockSpec((1, CPAD, CH), lambda w: (w, 0, 0))),
        out_shape=(jax.ShapeDtypeStruct((NW, PPAD, CH), jnp.float32),
                   jax.ShapeDtypeStruct((NW, CPAD, CH), jnp.float32)),
    )(poi_pad, cat_pad, Wp, Wc)


def _sc_gather(poi_proj, cat_proj, pidx, cidx):
    mesh = plsc.VectorSubcoreMesh(core_axis_name="c", subcore_axis_name="s")

    @functools.partial(
        pl.kernel,
        out_type=jax.ShapeDtypeStruct((B * NP, CH), jnp.float32),
        mesh=mesh,
        scratch_types=[
            pltpu.VMEM((RCH, 128), jnp.int32),
            pltpu.VMEM((RCH, 128), jnp.int32),
            pltpu.VMEM((128, CH), jnp.float32),
            pltpu.VMEM((128, CH), jnp.float32),
            pltpu.SemaphoreType.DMA,
            pltpu.SemaphoreType.DMA,
            pltpu.SemaphoreType.DMA,
        ],
    )
    def k(poi_hbm, cat_hbm, pidx_hbm, cidx_hbm, xin_hbm,
          pib, cib, prow, crow, sem0, sem1, semw):
        wid = lax.axis_index("c") * NS + lax.axis_index("s")
        pltpu.sync_copy(pidx_hbm.at[wid], pib)
        pltpu.sync_copy(cidx_hbm.at[wid], cib)

        def chunk(ch, _):
            base = wid * ROWS_W + ch * 128
            cp = pltpu.async_copy(poi_hbm.at[pib.at[ch]], prow, sem0)
            cc = pltpu.async_copy(cat_hbm.at[cib.at[ch]], crow, sem1)
            cp.wait()
            cc.wait()

            def row(r, _):
                for kk in range(8):
                    sl = pl.ds(kk * 16, 16)
                    prow[r, sl] = prow[r, sl] + crow[r, sl]
                return ()
            lax.fori_loop(0, 128, row, ())
            pltpu.async_copy(prow, xin_hbm.at[pl.ds(base, 128)], semw).wait()
            return ()
        lax.fori_loop(0, RCH, chunk, ())

    return k(poi_proj, cat_proj, pidx, cidx)


def _sc_build_adj(gidx, ew):
    mesh = plsc.VectorSubcoreMesh(core_axis_name="c", subcore_axis_name="s")
    GPC = B // NC

    @functools.partial(
        pl.kernel,
        out_type=(jax.ShapeDtypeStruct((B, NP, NP), jnp.float32),
                  jax.ShapeDtypeStruct((B, NP, NP), jnp.float32)),
        mesh=mesh,
        scratch_types=[
            pltpu.VMEM((ECH, 128), jnp.int32),
            pltpu.VMEM((ECH, 128), jnp.float32),
            pltpu.VMEM((128,), jnp.float32),
            pltpu.VMEM((TPS,), jnp.float32),
            pltpu.VMEM_SHARED((NPNP,), jnp.float32),
            pltpu.VMEM_SHARED((NPNP,), jnp.float32),
            pltpu.SemaphoreType.DMA,
            pltpu.SemaphoreType.DMA,
            pltpu.SemaphoreType.DMA,
        ],
    )
    def k(gidx_hbm, ew_hbm, outc_hbm, outw_hbm,
          ib, wb, ones, zb, accc, accw, sems, semd, seme):
        c = lax.axis_index("c")
        s = lax.axis_index("s")

        def fill16(ref, val, n):
            def bodyf(i, _):
                ref[pl.ds(i * 16, 16)] = jnp.full((16,), val, jnp.float32)
                return ()
            lax.fori_loop(0, n, bodyf, ())
        fill16(ones, 1.0, 8)
        fill16(zb, 0.0, TPS // 16)

        sl = pl.ds(s * TPS, TPS)
        pltpu.async_copy(zb, accc.at[sl], semd)
        pltpu.async_copy(zb, accw.at[sl], semd)
        cpe0 = pltpu.async_copy(gidx_hbm.at[c * GPC, s], ib, seme)
        cpe1 = pltpu.async_copy(ew_hbm.at[c * GPC, s], wb, seme)
        pltpu.make_async_copy(zb, accc.at[sl], semd).wait()
        pltpu.make_async_copy(zb, accw.at[sl], semd).wait()
        cpe0.wait()
        cpe1.wait()
        plsc.subcore_barrier()

        def per_graph(j, _):
            g = c * GPC + j
            with jax.named_scope("adj_scatter"):
                for i in range(ECH):
                    pltpu.async_copy(ones, accc.at[ib.at[i]], sems, add=True)
                    pltpu.async_copy(wb.at[i], accw.at[ib.at[i]], sems,
                                     add=True)
                for i in range(ECH):
                    pltpu.make_async_copy(ones, accc.at[ib.at[i]],
                                          sems).wait()
                    pltpu.make_async_copy(wb.at[i], accw.at[ib.at[i]],
                                          sems).wait()
            plsc.subcore_barrier()
            with jax.named_scope("adj_drain"):
                def drain_row(r, _):
                    rr = s * RPT + r
                    pltpu.async_copy(accc.at[pl.ds(rr * NP, NP)],
                                     outc_hbm.at[g, rr, :], semd)
                    pltpu.async_copy(accw.at[pl.ds(rr * NP, NP)],
                                     outw_hbm.at[g, rr, :], semd)
                    return ()
                lax.fori_loop(0, RPT, drain_row, ())

                def wait_row(r, _):
                    pltpu.make_async_copy(accc.at[pl.ds(0, NP)],
                                          outc_hbm.at[g, 0, :], semd).wait()
                    pltpu.make_async_copy(accw.at[pl.ds(0, NP)],
                                          outw_hbm.at[g, 0, :], semd).wait()
                    return ()
                lax.fori_loop(0, RPT, wait_row, ())

            @pl.when(j + 1 < GPC)
            def _():
                with jax.named_scope("adj_refill"):
                    ce0 = pltpu.async_copy(gidx_hbm.at[g + 1, s], ib, seme)
                    ce1 = pltpu.async_copy(ew_hbm.at[g + 1, s], wb, seme)
                    zc = pltpu.async_copy(zb, accc.at[sl], semd)
                    zw = pltpu.async_copy(zb, accw.at[sl], semd)
                    zc.wait()
                    zw.wait()
                    ce0.wait()
                    ce1.wait()
            plsc.subcore_barrier()
            return ()
        lax.fori_loop(0, GPC, per_graph, ())

    return k(gidx, ew)


def _tc_gnn(xine, f3, cnt, wsum, W3p, bin2, Wg_all, AS, AD, BG,
            Woutp, bo):
    def body(xp_ref, f3_ref, cnt_ref, ws_ref, w3_ref, bin_ref,
             wg_ref, as_ref, ad_ref, bg_ref, wo_ref, bo_ref, out_ref):
        cntm = cnt_ref[0]
        wsm = ws_ref[0]
        xin = (xp_ref[0]
               + jnp.dot(f3_ref[0], w3_ref[...],
                         preferred_element_type=jnp.float32))
        deg = jnp.sum(wsm, axis=1, keepdims=True) + 1.0
        dis = 1.0 / jnp.sqrt(deg)
        dis2 = dis * dis

        def gcn_apply(h):
            t = jnp.dot(wsm, h * dis, preferred_element_type=jnp.float32)
            return dis * t + dis2 * h

        f = _leaky(gcn_apply(xin) + bin_ref[0][None, :], 0.01)

        r = lax.broadcasted_iota(jnp.int32, (NP, NP), 0)
        cidx = lax.broadcasted_iota(jnp.int32, (NP, NP), 1)
        cpi = cntm + jnp.where(r == cidx, 1.0, 0.0)
        valid = cpi > 0

        def gat(x, i):
            h = jnp.dot(x, wg_ref[i], preferred_element_type=jnp.float32)
            asrc = jnp.sum(h * as_ref[i][None, :], axis=1)
            adst = jnp.sum(h * ad_ref[i][None, :], axis=1)
            e = adst[:, None] + asrc[None, :]
            e = _leaky(e, 0.2)
            m = jnp.max(jnp.where(valid, e, -3e38), axis=1, keepdims=True)
            p = cpi * jnp.exp(jnp.minimum(e - m, 0.0))
            den = jnp.sum(p, axis=1, keepdims=True)
            agg = jnp.dot(p, h, preferred_element_type=jnp.float32)
            return agg / den + bg_ref[i][None, :]

        for i in range(3):
            t = gat(f, i)
            y = _leaky(t, 0.01) + t
            y2 = gat(y, i)
            f = _leaky(y2, 0.01)

        h2 = jnp.dot(f, wo_ref[...], preferred_element_type=jnp.float32)
        f2 = _leaky(gcn_apply(h2) + bo_ref[0][None, :], 0.01)
        out_ref[0, 0] = f2[:, 0]

    grid = (B,)
    return pl.pallas_call(
        body,
        grid=grid,
        in_specs=[
            pl.BlockSpec((1, NP, CH), lambda g: (g, 0, 0)),
            pl.BlockSpec((1, NP, 8), lambda g: (g, 0, 0)),
            pl.BlockSpec((1, NP, NP), lambda g: (g, 0, 0)),
            pl.BlockSpec((1, NP, NP), lambda g: (g, 0, 0)),
            pl.BlockSpec((8, CH), lambda g: (0, 0)),
            pl.BlockSpec((1, CH), lambda g: (0, 0)),
            pl.BlockSpec((3, CH, CH), lambda g: (0, 0, 0)),
            pl.BlockSpec((8, CH), lambda g: (0, 0)),
            pl.BlockSpec((8, CH), lambda g: (0, 0)),
            pl.BlockSpec((8, CH), lambda g: (0, 0)),
            pl.BlockSpec((CH, 8), lambda g: (0, 0)),
            pl.BlockSpec((1, 8), lambda g: (0, 0)),
        ],
        out_specs=pl.BlockSpec((1, 1, NP), lambda g: (g, 0, 0)),
        out_shape=jax.ShapeDtypeStruct((B, 1, NP), jnp.float32),
    )(xine, f3, cnt, wsum, W3p, bin2, Wg_all, AS, AD, BG, Woutp, bo)


def _tc_head(fgr, W1p, b1, W2p, b2p):
    def body(f_ref, w1_ref, b1_ref, w2_ref, b2_ref, out_ref):
        h = jnp.maximum(
            jnp.dot(f_ref[...], w1_ref[...],
                    preferred_element_type=jnp.float32) + b1_ref[...], 0.0)
        out_ref[...] = jnp.maximum(
            jnp.dot(h, w2_ref[...],
                    preferred_element_type=jnp.float32) + b2_ref[...], 0.0)

    return pl.pallas_call(
        body,
        out_shape=jax.ShapeDtypeStruct((B, OPAD), jnp.float32),
    )(fgr, W1p, b1, W2p, b2p)


def kernel(feature, edges, weight, params):
    f32 = jnp.float32

    poi_idx = feature[:, :, 0].astype(jnp.int32)
    cat_idx = feature[:, :, 1].astype(jnp.int32)

    nar = jnp.arange(NP, dtype=jnp.int32)[None, :]
    ppad = PLEN + nar % (PPAD - PLEN)
    cpad = CLEN + nar % (CPAD - CLEN)
    pidx = jnp.broadcast_to(ppad, (B, NP)).at[:, :NODE].set(poi_idx)
    cidx = jnp.broadcast_to(cpad, (B, NP)).at[:, :NODE].set(cat_idx)
    woff = jnp.arange(NW, dtype=jnp.int32)[:, None, None]
    pidx = pidx.reshape(NW, RCH, 128) + woff * PPAD
    cidx = cidx.reshape(NW, RCH, 128) + woff * CPAD

    src = edges[:, 0, :]
    dst = edges[:, 1, :]
    gidx = (dst * NP + src).reshape(B, NS, ECH, 128)
    ew = weight[:, :, 1].reshape(B, NS, ECH, 128).astype(f32)

    W_in = params['W_in']
    poi_pad = jnp.zeros((PPAD, 304), f32).at[:PLEN, :PDIM].set(
        params['poi_table'])
    Wp = jnp.zeros((304, CH), f32).at[:PDIM].set(W_in[:PDIM])
    cat_pad = jnp.zeros((CPAD, 112), f32).at[:CLEN, :CDIM].set(
        params['cat_table'])
    Wc = jnp.zeros((112, CH), f32).at[:CDIM].set(W_in[PDIM:PDIM + CDIM])

    f3 = jnp.zeros((B, NP, 8), f32).at[:, :NODE, 0:3].set(feature[:, :, 2:5])
    W3p = jnp.zeros((8, CH), f32).at[0:3].set(W_in[PDIM + CDIM:])
    bin2 = params['b_in'][None, :]

    Wg_all = jnp.stack([params['Wg%d' % i] for i in range(3)])
    AS = jnp.zeros((8, CH), f32).at[0:3].set(
        jnp.stack([params['as%d' % i] for i in range(3)]))
    AD = jnp.zeros((8, CH), f32).at[0:3].set(
        jnp.stack([params['ad%d' % i] for i in range(3)]))
    BG = jnp.zeros((8, CH), f32).at[0:3].set(
        jnp.stack([params['bg%d' % i] for i in range(3)]))

    Woutp = jnp.zeros((CH, 8), f32).at[:, 0].set(params['W_out'][:, 0])
    bo = jnp.broadcast_to(params['b_out'][0], (1, 8))

    W1p = jnp.zeros((NP, CH), f32).at[:NODE].set(params['W1'])
    b1 = params['b1'][None, :]
    W2p = jnp.zeros((CH, OPAD), f32).at[:, :PLEN].set(params['W2'])
    b2p = jnp.zeros((1, OPAD), f32).at[0, :PLEN].set(params['b2'])

    poi_proj, cat_proj = _tc_proj(poi_pad, cat_pad, Wp, Wc)
    xine = _sc_gather(poi_proj.reshape(NW * PPAD, CH),
                      cat_proj.reshape(NW * CPAD, CH),
                      pidx, cidx).reshape(B, NP, CH)
    cnt, wsum = _sc_build_adj(gidx, ew)

    fgr = _tc_gnn(xine, f3, cnt, wsum, W3p, bin2, Wg_all, AS, AD, BG,
                  Woutp, bo).reshape(B, NP)
    out = _tc_head(fgr, W1p, b1, W2p, b2p)
    return out[:, :PLEN]


# speedup vs baseline: 102.2445x; 1.0511x over previous
import functools

import jax
import jax.numpy as jnp
from jax import lax
from jax.experimental import pallas as pl
from jax.experimental.pallas import tpu as pltpu
from jax.experimental.pallas import tpu_sc as plsc

B = 64
NODE = 714
EPG = 8192
PLEN = 5099
CLEN = 400
PDIM = 300
CDIM = 100
CH = 128

NP = 768
NPNP = NP * NP
NC, NS = 2, 16
NW = NC * NS
TPS = NPNP // NS
RPT = NP // NS
EPT = EPG // NS
ECH = EPT // 128
ROWS_W = (B * NP) // NW
RCH = ROWS_W // 128

PPAD = 5104
CPAD = 408
OPAD = 5120


def _leaky(x, slope):
    return jnp.maximum(x, slope * x)


def _tc_proj(poi_pad, cat_pad, Wp, Wc):
    def body(poi_ref, cat_ref, wp_ref, wc_ref, op_ref, oc_ref):
        op_ref[0] = jnp.dot(poi_ref[...], wp_ref[...],
                            preferred_element_type=jnp.float32)
        oc_ref[0] = jnp.dot(cat_ref[...], wc_ref[...],
                            preferred_element_type=jnp.float32)

    return pl.pallas_call(
        body,
        grid=(NW,),
        in_specs=[
            pl.BlockSpec((PPAD, 304), lambda w: (0, 0)),
            pl.BlockSpec((CPAD, 112), lambda w: (0, 0)),
            pl.BlockSpec((304, CH), lambda w: (0, 0)),
            pl.BlockSpec((112, CH), lambda w: (0, 0)),
        ],
        out_specs=(pl.BlockSpec((1, PPAD, CH), lambda w: (w, 0, 0)),
                   pl.BlockSpec((1, CPAD, CH), lambda w: (w, 0, 0))),
        out_shape=(jax.ShapeDtypeStruct((NW, PPAD, CH), jnp.float32),
                   jax.ShapeDtypeStruct((NW, CPAD, CH), jnp.float32)),
    )(poi_pad, cat_pad, Wp, Wc)


def _sc_gather(poi_proj, cat_proj, pidx, cidx):
    mesh = plsc.VectorSubcoreMesh(core_axis_name="c", subcore_axis_name="s")

    @functools.partial(
        pl.kernel,
        out_type=jax.ShapeDtypeStruct((B * NP, CH), jnp.float32),
        mesh=mesh,
        scratch_types=[
            pltpu.VMEM((RCH, 128), jnp.int32),
            pltpu.VMEM((RCH, 128), jnp.int32),
            pltpu.VMEM((128, CH), jnp.float32),
            pltpu.VMEM((128, CH), jnp.float32),
            pltpu.SemaphoreType.DMA,
            pltpu.SemaphoreType.DMA,
            pltpu.SemaphoreType.DMA,
        ],
    )
    def k(poi_hbm, cat_hbm, pidx_hbm, cidx_hbm, xin_hbm,
          pib, cib, prow, crow, sem0, sem1, semw):
        wid = lax.axis_index("c") * NS + lax.axis_index("s")
        pltpu.sync_copy(pidx_hbm.at[wid], pib)
        pltpu.sync_copy(cidx_hbm.at[wid], cib)

        def chunk(ch, _):
            base = wid * ROWS_W + ch * 128
            cp = pltpu.async_copy(poi_hbm.at[pib.at[ch]], prow, sem0)
            cc = pltpu.async_copy(cat_hbm.at[cib.at[ch]], crow, sem1)
            cp.wait()
            cc.wait()

            def row(r, _):
                for kk in range(8):
                    sl = pl.ds(kk * 16, 16)
                    prow[r, sl] = prow[r, sl] + crow[r, sl]
                return ()
            lax.fori_loop(0, 128, row, ())
            pltpu.async_copy(prow, xin_hbm.at[pl.ds(base, 128)], semw).wait()
            return ()
        lax.fori_loop(0, RCH, chunk, ())

    return k(poi_proj, cat_proj, pidx, cidx)


def _sc_build_adj(gidx, ew):
    mesh = plsc.VectorSubcoreMesh(core_axis_name="c", subcore_axis_name="s")
    GPC = B // NC

    @functools.partial(
        pl.kernel,
        out_type=(jax.ShapeDtypeStruct((B, NP, NP), jnp.float32),
                  jax.ShapeDtypeStruct((B, NP, NP), jnp.float32)),
        mesh=mesh,
        scratch_types=[
            pltpu.VMEM((ECH, 128), jnp.int32),
            pltpu.VMEM((ECH, 128), jnp.float32),
            pltpu.VMEM((128,), jnp.float32),
            pltpu.VMEM((TPS,), jnp.float32),
            pltpu.VMEM_SHARED((NPNP,), jnp.float32),
            pltpu.VMEM_SHARED((NPNP,), jnp.float32),
            pltpu.SemaphoreType.DMA,
            pltpu.SemaphoreType.DMA,
            pltpu.SemaphoreType.DMA,
        ],
    )
    def k(gidx_hbm, ew_hbm, outc_hbm, outw_hbm,
          ib, wb, ones, zb, accc, accw, sems, semd, seme):
        c = lax.axis_index("c")
        s = lax.axis_index("s")

        def fill16(ref, val, n):
            def bodyf(i, _):
                ref[pl.ds(i * 16, 16)] = jnp.full((16,), val, jnp.float32)
                return ()
            lax.fori_loop(0, n, bodyf, ())
        fill16(ones, 1.0, 8)
        fill16(zb, 0.0, TPS // 16)

        sl = pl.ds(s * TPS, TPS)
        pltpu.async_copy(zb, accc.at[sl], semd)
        pltpu.async_copy(zb, accw.at[sl], semd)
        cpe0 = pltpu.async_copy(gidx_hbm.at[c * GPC, s], ib, seme)
        cpe1 = pltpu.async_copy(ew_hbm.at[c * GPC, s], wb, seme)
        pltpu.make_async_copy(zb, accc.at[sl], semd).wait()
        pltpu.make_async_copy(zb, accw.at[sl], semd).wait()
        cpe0.wait()
        cpe1.wait()
        plsc.subcore_barrier()

        def per_graph(j, _):
            g = c * GPC + j
            with jax.named_scope("adj_scatter"):
                for i in range(ECH):
                    pltpu.async_copy(ones, accc.at[ib.at[i]], sems, add=True)
                    pltpu.async_copy(wb.at[i], accw.at[ib.at[i]], sems,
                                     add=True)
                for i in range(ECH):
                    pltpu.make_async_copy(ones, accc.at[ib.at[i]],
                                          sems).wait()
                    pltpu.make_async_copy(wb.at[i], accw.at[ib.at[i]],
                                          sems).wait()
            plsc.subcore_barrier()
            with jax.named_scope("adj_drain"):
                def drain_row(r, _):
                    rr = s * RPT + r
                    pltpu.async_copy(accc.at[pl.ds(rr * NP, NP)],
                                     outc_hbm.at[g, rr, :], semd)
                    pltpu.async_copy(accw.at[pl.ds(rr * NP, NP)],
                                     outw_hbm.at[g, rr, :], semd)
                    return ()
                lax.fori_loop(0, RPT, drain_row, ())

                def wait_row(r, _):
                    pltpu.make_async_copy(accc.at[pl.ds(0, NP)],
                                          outc_hbm.at[g, 0, :], semd).wait()
                    pltpu.make_async_copy(accw.at[pl.ds(0, NP)],
                                          outw_hbm.at[g, 0, :], semd).wait()
                    return ()
                lax.fori_loop(0, RPT, wait_row, ())

            @pl.when(j + 1 < GPC)
            def _():
                with jax.named_scope("adj_refill"):
                    ce0 = pltpu.async_copy(gidx_hbm.at[g + 1, s], ib, seme)
                    ce1 = pltpu.async_copy(ew_hbm.at[g + 1, s], wb, seme)
                    zc = pltpu.async_copy(zb, accc.at[sl], semd)
                    zw = pltpu.async_copy(zb, accw.at[sl], semd)
                    zc.wait()
                    zw.wait()
                    ce0.wait()
                    ce1.wait()
            plsc.subcore_barrier()
            return ()
        lax.fori_loop(0, GPC, per_graph, ())

    return k(gidx, ew)


def _tc_gnn(xine, f3, cnt, wsum, W3p, bin2, Wg_all, AA, BG,
            Woutp, bo):
    def body(xp_ref, f3_ref, cnt_ref, ws_ref, w3_ref, bin_ref,
             wg_ref, aa_ref, bg_ref, wo_ref, bo_ref, out_ref):
        cntm = cnt_ref[0]
        wsm = ws_ref[0]
        xin = (xp_ref[0]
               + jnp.dot(f3_ref[0], w3_ref[...],
                         preferred_element_type=jnp.float32))
        deg = jnp.sum(wsm, axis=1, keepdims=True) + 1.0
        dis = 1.0 / jnp.sqrt(deg)
        dis2 = dis * dis

        def gcn_apply(h):
            t = jnp.dot(wsm, h * dis, preferred_element_type=jnp.float32)
            return dis * t + dis2 * h

        f = _leaky(gcn_apply(xin) + bin_ref[0][None, :], 0.01)

        r = lax.broadcasted_iota(jnp.int32, (NP, NP), 0)
        cidx = lax.broadcasted_iota(jnp.int32, (NP, NP), 1)
        cpi = cntm + jnp.where(r == cidx, 1.0, 0.0)
        ones_col = jnp.zeros((NP, 8), jnp.float32) + jnp.where(
            lax.broadcasted_iota(jnp.int32, (NP, 8), 1) == 0, 1.0, 0.0)

        def gat(x, i):
            h = jnp.dot(x, wg_ref[i], preferred_element_type=jnp.float32)
            ha = jnp.dot(h, aa_ref[i], preferred_element_type=jnp.float32)
            asrc = ha[:, 0]
            adst = ha[:, 1]
            e = adst[:, None] + asrc[None, :]
            e = jnp.maximum(e, 0.2 * e)
            m = jnp.max(e, axis=1, keepdims=True)
            p = cpi * jnp.exp(e - m)
            den = jnp.dot(p, ones_col,
                          preferred_element_type=jnp.float32)[:, 0:1]
            agg = jnp.dot(p, h, preferred_element_type=jnp.float32)
            return agg / den + bg_ref[i][None, :]

        for i in range(3):
            t = gat(f, i)
            y = _leaky(t, 0.01) + t
            y2 = gat(y, i)
            f = _leaky(y2, 0.01)

        h2 = jnp.dot(f, wo_ref[...], preferred_element_type=jnp.float32)
        f2 = _leaky(gcn_apply(h2) + bo_ref[0][None, :], 0.01)
        out_ref[0, 0] = f2[:, 0]

    grid = (B,)
    return pl.pallas_call(
        body,
        grid=grid,
        in_specs=[
            pl.BlockSpec((1, NP, CH), lambda g: (g, 0, 0)),
            pl.BlockSpec((1, NP, 8), lambda g: (g, 0, 0)),
            pl.BlockSpec((1, NP, NP), lambda g: (g, 0, 0)),
            pl.BlockSpec((1, NP, NP), lambda g: (g, 0, 0)),
            pl.BlockSpec((8, CH), lambda g: (0, 0)),
            pl.BlockSpec((1, CH), lambda g: (0, 0)),
            pl.BlockSpec((3, CH, CH), lambda g: (0, 0, 0)),
            pl.BlockSpec((3, CH, 8), lambda g: (0, 0, 0)),
            pl.BlockSpec((8, CH), lambda g: (0, 0)),
            pl.BlockSpec((CH, 8), lambda g: (0, 0)),
            pl.BlockSpec((1, 8), lambda g: (0, 0)),
        ],
        out_specs=pl.BlockSpec((1, 1, NP), lambda g: (g, 0, 0)),
        out_shape=jax.ShapeDtypeStruct((B, 1, NP), jnp.float32),
    )(xine, f3, cnt, wsum, W3p, bin2, Wg_all, AA, BG, Woutp, bo)


def _tc_head(fgr, W1p, b1, W2p, b2p):
    def body(f_ref, w1_ref, b1_ref, w2_ref, b2_ref, out_ref):
        h = jnp.maximum(
            jnp.dot(f_ref[...], w1_ref[...],
                    preferred_element_type=jnp.float32) + b1_ref[...], 0.0)
        out_ref[...] = jnp.maximum(
            jnp.dot(h, w2_ref[...],
                    preferred_element_type=jnp.float32) + b2_ref[...], 0.0)

    return pl.pallas_call(
        body,
        out_shape=jax.ShapeDtypeStruct((B, OPAD), jnp.float32),
    )(fgr, W1p, b1, W2p, b2p)


def kernel(feature, edges, weight, params):
    f32 = jnp.float32

    poi_idx = feature[:, :, 0].astype(jnp.int32)
    cat_idx = feature[:, :, 1].astype(jnp.int32)

    nar = jnp.arange(NP, dtype=jnp.int32)[None, :]
    ppad = PLEN + nar % (PPAD - PLEN)
    cpad = CLEN + nar % (CPAD - CLEN)
    pidx = jnp.broadcast_to(ppad, (B, NP)).at[:, :NODE].set(poi_idx)
    cidx = jnp.broadcast_to(cpad, (B, NP)).at[:, :NODE].set(cat_idx)
    woff = jnp.arange(NW, dtype=jnp.int32)[:, None, None]
    pidx = pidx.reshape(NW, RCH, 128) + woff * PPAD
    cidx = cidx.reshape(NW, RCH, 128) + woff * CPAD

    src = edges[:, 0, :]
    dst = edges[:, 1, :]
    gidx = (dst * NP + src).reshape(B, NS, ECH, 128)
    ew = weight[:, :, 1].reshape(B, NS, ECH, 128).astype(f32)

    W_in = params['W_in']
    poi_pad = jnp.zeros((PPAD, 304), f32).at[:PLEN, :PDIM].set(
        params['poi_table'])
    Wp = jnp.zeros((304, CH), f32).at[:PDIM].set(W_in[:PDIM])
    cat_pad = jnp.zeros((CPAD, 112), f32).at[:CLEN, :CDIM].set(
        params['cat_table'])
    Wc = jnp.zeros((112, CH), f32).at[:CDIM].set(W_in[PDIM:PDIM + CDIM])

    f3 = jnp.zeros((B, NP, 8), f32).at[:, :NODE, 0:3].set(feature[:, :, 2:5])
    W3p = jnp.zeros((8, CH), f32).at[0:3].set(W_in[PDIM + CDIM:])
    bin2 = params['b_in'][None, :]

    Wg_all = jnp.stack([params['Wg%d' % i] for i in range(3)])
    AA = jnp.zeros((3, CH, 8), f32)
    AA = AA.at[:, :, 0].set(jnp.stack([params['as%d' % i] for i in range(3)]))
    AA = AA.at[:, :, 1].set(jnp.stack([params['ad%d' % i] for i in range(3)]))
    BG = jnp.zeros((8, CH), f32).at[0:3].set(
        jnp.stack([params['bg%d' % i] for i in range(3)]))

    Woutp = jnp.zeros((CH, 8), f32).at[:, 0].set(params['W_out'][:, 0])
    bo = jnp.broadcast_to(params['b_out'][0], (1, 8))

    W1p = jnp.zeros((NP, CH), f32).at[:NODE].set(params['W1'])
    b1 = params['b1'][None, :]
    W2p = jnp.zeros((CH, OPAD), f32).at[:, :PLEN].set(params['W2'])
    b2p = jnp.zeros((1, OPAD), f32).at[0, :PLEN].set(params['b2'])

    poi_proj, cat_proj = _tc_proj(poi_pad, cat_pad, Wp, Wc)
    xine = _sc_gather(poi_proj.reshape(NW * PPAD, CH),
                      cat_proj.reshape(NW * CPAD, CH),
                      pidx, cidx).reshape(B, NP, CH)
    cnt, wsum = _sc_build_adj(gidx, ew)

    fgr = _tc_gnn(xine, f3, cnt, wsum, W3p, bin2, Wg_all, AA, BG,
                  Woutp, bo).reshape(B, NP)
    out = _tc_head(fgr, W1p, b1, W2p, b2p)
    return out[:, :PLEN]


# speedup vs baseline: 111.1831x; 1.0874x over previous
import functools

import jax
import jax.numpy as jnp
from jax import lax
from jax.experimental import pallas as pl
from jax.experimental.pallas import tpu as pltpu
from jax.experimental.pallas import tpu_sc as plsc

B = 64
NODE = 714
EPG = 8192
PLEN = 5099
CLEN = 400
PDIM = 300
CDIM = 100
CH = 128

NP = 768
NPNP = NP * NP
NC, NS = 2, 16
NW = NC * NS
TPS = NPNP // NS
RPT = NP // NS
EPT = EPG // NS
ECH = EPT // 128
ROWS_W = (B * NP) // NW
RCH = ROWS_W // 128

PPAD = 5104
CPAD = 408
OPAD = 5120


def _leaky(x, slope):
    return jnp.maximum(x, slope * x)


def _tc_proj(poi_pad, cat_pad, Wp, Wc):
    def body(poi_ref, cat_ref, wp_ref, wc_ref, op_ref, oc_ref):
        op_ref[0] = jnp.dot(poi_ref[...], wp_ref[...],
                            preferred_element_type=jnp.float32)
        oc_ref[0] = jnp.dot(cat_ref[...], wc_ref[...],
                            preferred_element_type=jnp.float32)

    return pl.pallas_call(
        body,
        grid=(NW,),
        in_specs=[
            pl.BlockSpec((PPAD, 304), lambda w: (0, 0)),
            pl.BlockSpec((CPAD, 112), lambda w: (0, 0)),
            pl.BlockSpec((304, CH), lambda w: (0, 0)),
            pl.BlockSpec((112, CH), lambda w: (0, 0)),
        ],
        out_specs=(pl.BlockSpec((1, PPAD, CH), lambda w: (w, 0, 0)),
                   pl.BlockSpec((1, CPAD, CH), lambda w: (w, 0, 0))),
        out_shape=(jax.ShapeDtypeStruct((NW, PPAD, CH), jnp.float32),
                   jax.ShapeDtypeStruct((NW, CPAD, CH), jnp.float32)),
    )(poi_pad, cat_pad, Wp, Wc)


def _sc_gather(poi_proj, cat_proj, pidx, cidx):
    mesh = plsc.VectorSubcoreMesh(core_axis_name="c", subcore_axis_name="s")

    @functools.partial(
        pl.kernel,
        out_type=jax.ShapeDtypeStruct((B * NP, CH), jnp.float32),
        mesh=mesh,
        scratch_types=[
            pltpu.VMEM((RCH, 128), jnp.int32),
            pltpu.VMEM((RCH, 128), jnp.int32),
            pltpu.VMEM((128, CH), jnp.float32),
            pltpu.VMEM((128, CH), jnp.float32),
            pltpu.SemaphoreType.DMA,
            pltpu.SemaphoreType.DMA,
            pltpu.SemaphoreType.DMA,
        ],
    )
    def k(poi_hbm, cat_hbm, pidx_hbm, cidx_hbm, xin_hbm,
          pib, cib, prow, crow, sem0, sem1, semw):
        wid = lax.axis_index("c") * NS + lax.axis_index("s")
        pltpu.sync_copy(pidx_hbm.at[wid], pib)
        pltpu.sync_copy(cidx_hbm.at[wid], cib)

        def chunk(ch, _):
            base = wid * ROWS_W + ch * 128
            cp = pltpu.async_copy(poi_hbm.at[pib.at[ch]], prow, sem0)
            cc = pltpu.async_copy(cat_hbm.at[cib.at[ch]], crow, sem1)
            cp.wait()
            cc.wait()

            def row(r, _):
                for kk in range(8):
                    sl = pl.ds(kk * 16, 16)
                    prow[r, sl] = prow[r, sl] + crow[r, sl]
                return ()
            lax.fori_loop(0, 128, row, ())
            pltpu.async_copy(prow, xin_hbm.at[pl.ds(base, 128)], semw).wait()
            return ()
        lax.fori_loop(0, RCH, chunk, ())

    return k(poi_proj, cat_proj, pidx, cidx)


def _sc_build_adj(gidx, ew):
    mesh = plsc.VectorSubcoreMesh(core_axis_name="c", subcore_axis_name="s")
    GPC = B // NC

    @functools.partial(
        pl.kernel,
        out_type=(jax.ShapeDtypeStruct((B, NP, NP), jnp.float32),
                  jax.ShapeDtypeStruct((B, NP, NP), jnp.float32)),
        mesh=mesh,
        scratch_types=[
            pltpu.VMEM((ECH, 128), jnp.int32),
            pltpu.VMEM((ECH, 128), jnp.float32),
            pltpu.VMEM((128,), jnp.float32),
            pltpu.VMEM((TPS,), jnp.float32),
            pltpu.VMEM_SHARED((NPNP,), jnp.float32),
            pltpu.VMEM_SHARED((NPNP,), jnp.float32),
            pltpu.SemaphoreType.DMA,
            pltpu.SemaphoreType.DMA,
            pltpu.SemaphoreType.DMA,
        ],
    )
    def k(gidx_hbm, ew_hbm, outc_hbm, outw_hbm,
          ib, wb, ones, zb, accc, accw, sems, semd, seme):
        c = lax.axis_index("c")
        s = lax.axis_index("s")

        def fill16(ref, val, n):
            def bodyf(i, _):
                ref[pl.ds(i * 16, 16)] = jnp.full((16,), val, jnp.float32)
                return ()
            lax.fori_loop(0, n, bodyf, ())
        fill16(ones, 1.0, 8)
        fill16(zb, 0.0, TPS // 16)

        sl = pl.ds(s * TPS, TPS)
        pltpu.async_copy(zb, accc.at[sl], semd)
        pltpu.async_copy(zb, accw.at[sl], semd)
        cpe0 = pltpu.async_copy(gidx_hbm.at[c * GPC, s], ib, seme)
        cpe1 = pltpu.async_copy(ew_hbm.at[c * GPC, s], wb, seme)
        pltpu.make_async_copy(zb, accc.at[sl], semd).wait()
        pltpu.make_async_copy(zb, accw.at[sl], semd).wait()
        cpe0.wait()
        cpe1.wait()
        plsc.subcore_barrier()

        def per_graph(j, _):
            g = c * GPC + j
            with jax.named_scope("adj_scatter"):
                for i in range(ECH):
                    pltpu.async_copy(ones, accc.at[ib.at[i]], sems, add=True)
                    pltpu.async_copy(wb.at[i], accw.at[ib.at[i]], sems,
                                     add=True)
                for i in range(ECH):
                    pltpu.make_async_copy(ones, accc.at[ib.at[i]],
                                          sems).wait()
                    pltpu.make_async_copy(wb.at[i], accw.at[ib.at[i]],
                                          sems).wait()
            plsc.subcore_barrier()
            with jax.named_scope("adj_drain"):
                def drain_row(r, _):
                    rr = s * RPT + r
                    pltpu.async_copy(accc.at[pl.ds(rr * NP, NP)],
                                     outc_hbm.at[g, rr, :], semd)
                    pltpu.async_copy(accw.at[pl.ds(rr * NP, NP)],
                                     outw_hbm.at[g, rr, :], semd)
                    return ()
                lax.fori_loop(0, RPT, drain_row, ())

                def wait_row(r, _):
                    pltpu.make_async_copy(accc.at[pl.ds(0, NP)],
                                          outc_hbm.at[g, 0, :], semd).wait()
                    pltpu.make_async_copy(accw.at[pl.ds(0, NP)],
                                          outw_hbm.at[g, 0, :], semd).wait()
                    return ()
                lax.fori_loop(0, RPT, wait_row, ())

            @pl.when(j + 1 < GPC)
            def _():
                with jax.named_scope("adj_refill"):
                    ce0 = pltpu.async_copy(gidx_hbm.at[g + 1, s], ib, seme)
                    ce1 = pltpu.async_copy(ew_hbm.at[g + 1, s], wb, seme)
                    zc = pltpu.async_copy(zb, accc.at[sl], semd)
                    zw = pltpu.async_copy(zb, accw.at[sl], semd)
                    zc.wait()
                    zw.wait()
                    ce0.wait()
                    ce1.wait()
            plsc.subcore_barrier()
            return ()
        lax.fori_loop(0, GPC, per_graph, ())

    return k(gidx, ew)


def _tc_gnn(xine, f3, cnt, wsum, W3p, bin2, Wgx, BG,
            Woutp, bo):
    def body(xp_ref, f3_ref, cnt_ref, ws_ref, w3_ref, bin_ref,
             wg_ref, bg_ref, wo_ref, bo_ref, out_ref):
        cntm = cnt_ref[0]
        wsm = ws_ref[0]
        xin = (xp_ref[0]
               + jnp.dot(f3_ref[0], w3_ref[...],
                         preferred_element_type=jnp.float32))
        deg = jnp.sum(wsm, axis=1, keepdims=True) + 1.0
        dis = 1.0 / jnp.sqrt(deg)
        dis2 = dis * dis

        def gcn_apply(h):
            t = jnp.dot(wsm, h * dis, preferred_element_type=jnp.float32)
            return dis * t + dis2 * h

        f = _leaky(gcn_apply(xin) + bin_ref[0][None, :], 0.01)

        r = lax.broadcasted_iota(jnp.int32, (NP, NP), 0)
        cidx = lax.broadcasted_iota(jnp.int32, (NP, NP), 1)
        cpi = cntm + jnp.where(r == cidx, 1.0, 0.0)
        lane136 = lax.broadcasted_iota(jnp.int32, (NP, 136), 1)

        def gat(x, i):
            hx = jnp.dot(x, wg_ref[i], preferred_element_type=jnp.float32)
            asrc = hx[:, 128]
            adst = hx[:, 129]
            e = adst[:, None] + asrc[None, :]
            e = jnp.maximum(e, 0.2 * e)
            m = jnp.max(e, axis=1, keepdims=True)
            p = cpi * jnp.exp(e - m)
            hext = jnp.where(lane136 == 130, 1.0, hx)
            agg2 = jnp.dot(p, hext, preferred_element_type=jnp.float32)
            den = agg2[:, 130:131]
            return agg2[:, :CH] / den + bg_ref[i][None, :]

        for i in range(3):
            t = gat(f, i)
            y = _leaky(t, 0.01) + t
            y2 = gat(y, i)
            f = _leaky(y2, 0.01)

        h2 = jnp.dot(f, wo_ref[...], preferred_element_type=jnp.float32)
        f2 = _leaky(gcn_apply(h2) + bo_ref[0][None, :], 0.01)
        out_ref[0, 0] = f2[:, 0]

    grid = (B,)
    return pl.pallas_call(
        body,
        grid=grid,
        in_specs=[
            pl.BlockSpec((1, NP, CH), lambda g: (g, 0, 0)),
            pl.BlockSpec((1, NP, 8), lambda g: (g, 0, 0)),
            pl.BlockSpec((1, NP, NP), lambda g: (g, 0, 0)),
            pl.BlockSpec((1, NP, NP), lambda g: (g, 0, 0)),
            pl.BlockSpec((8, CH), lambda g: (0, 0)),
            pl.BlockSpec((1, CH), lambda g: (0, 0)),
            pl.BlockSpec((3, CH, 136), lambda g: (0, 0, 0)),
            pl.BlockSpec((8, CH), lambda g: (0, 0)),
            pl.BlockSpec((CH, 8), lambda g: (0, 0)),
            pl.BlockSpec((1, 8), lambda g: (0, 0)),
        ],
        out_specs=pl.BlockSpec((1, 1, NP), lambda g: (g, 0, 0)),
        out_shape=jax.ShapeDtypeStruct((B, 1, NP), jnp.float32),
    )(xine, f3, cnt, wsum, W3p, bin2, Wgx, BG, Woutp, bo)


def _tc_head(fgr, W1p, b1, W2p, b2p):
    def body(f_ref, w1_ref, b1_ref, w2_ref, b2_ref, out_ref):
        h = jnp.maximum(
            jnp.dot(f_ref[...], w1_ref[...],
                    preferred_element_type=jnp.float32) + b1_ref[...], 0.0)
        out_ref[...] = jnp.maximum(
            jnp.dot(h, w2_ref[...],
                    preferred_element_type=jnp.float32) + b2_ref[...], 0.0)

    return pl.pallas_call(
        body,
        out_shape=jax.ShapeDtypeStruct((B, OPAD), jnp.float32),
    )(fgr, W1p, b1, W2p, b2p)


def kernel(feature, edges, weight, params):
    f32 = jnp.float32

    poi_idx = feature[:, :, 0].astype(jnp.int32)
    cat_idx = feature[:, :, 1].astype(jnp.int32)

    nar = jnp.arange(NP, dtype=jnp.int32)[None, :]
    ppad = PLEN + nar % (PPAD - PLEN)
    cpad = CLEN + nar % (CPAD - CLEN)
    pidx = jnp.broadcast_to(ppad, (B, NP)).at[:, :NODE].set(poi_idx)
    cidx = jnp.broadcast_to(cpad, (B, NP)).at[:, :NODE].set(cat_idx)
    woff = jnp.arange(NW, dtype=jnp.int32)[:, None, None]
    pidx = pidx.reshape(NW, RCH, 128) + woff * PPAD
    cidx = cidx.reshape(NW, RCH, 128) + woff * CPAD

    src = edges[:, 0, :]
    dst = edges[:, 1, :]
    gidx = (dst * NP + src).reshape(B, NS, ECH, 128)
    ew = weight[:, :, 1].reshape(B, NS, ECH, 128).astype(f32)

    W_in = params['W_in']
    poi_pad = jnp.zeros((PPAD, 304), f32).at[:PLEN, :PDIM].set(
        params['poi_table'])
    Wp = jnp.zeros((304, CH), f32).at[:PDIM].set(W_in[:PDIM])
    cat_pad = jnp.zeros((CPAD, 112), f32).at[:CLEN, :CDIM].set(
        params['cat_table'])
    Wc = jnp.zeros((112, CH), f32).at[:CDIM].set(W_in[PDIM:PDIM + CDIM])

    f3 = jnp.zeros((B, NP, 8), f32).at[:, :NODE, 0:3].set(feature[:, :, 2:5])
    W3p = jnp.zeros((8, CH), f32).at[0:3].set(W_in[PDIM + CDIM:])
    bin2 = params['b_in'][None, :]

    Wg_all = jnp.stack([params['Wg%d' % i] for i in range(3)])
    Wgx = jnp.zeros((3, CH, 136), f32)
    Wgx = Wgx.at[:, :, :CH].set(Wg_all)
    Wgx = Wgx.at[:, :, CH].set(jnp.einsum(
        'ikc,ic->ik', Wg_all, jnp.stack(
            [params['as%d' % i] for i in range(3)])))
    Wgx = Wgx.at[:, :, CH + 1].set(jnp.einsum(
        'ikc,ic->ik', Wg_all, jnp.stack(
            [params['ad%d' % i] for i in range(3)])))
    BG = jnp.zeros((8, CH), f32).at[0:3].set(
        jnp.stack([params['bg%d' % i] for i in range(3)]))

    Woutp = jnp.zeros((CH, 8), f32).at[:, 0].set(params['W_out'][:, 0])
    bo = jnp.broadcast_to(params['b_out'][0], (1, 8))

    W1p = jnp.zeros((NP, CH), f32).at[:NODE].set(params['W1'])
    b1 = params['b1'][None, :]
    W2p = jnp.zeros((CH, OPAD), f32).at[:, :PLEN].set(params['W2'])
    b2p = jnp.zeros((1, OPAD), f32).at[0, :PLEN].set(params['b2'])

    poi_proj, cat_proj = _tc_proj(poi_pad, cat_pad, Wp, Wc)
    xine = _sc_gather(poi_proj.reshape(NW * PPAD, CH),
                      cat_proj.reshape(NW * CPAD, CH),
                      pidx, cidx).reshape(B, NP, CH)
    cnt, wsum = _sc_build_adj(gidx, ew)

    fgr = _tc_gnn(xine, f3, cnt, wsum, W3p, bin2, Wgx, BG,
                  Woutp, bo).reshape(B, NP)
    out = _tc_head(fgr, W1p, b1, W2p, b2p)
    return out[:, :PLEN]


# speedup vs baseline: 126.2009x; 1.1351x over previous
import functools

import jax
import jax.numpy as jnp
from jax import lax
from jax.experimental import pallas as pl
from jax.experimental.pallas import tpu as pltpu
from jax.experimental.pallas import tpu_sc as plsc

B = 64
NODE = 714
EPG = 8192
PLEN = 5099
CLEN = 400
PDIM = 300
CDIM = 100
CH = 128

NP = 768
NPNP = NP * NP
NC, NS = 2, 16
NW = NC * NS
TPS = NPNP // NS
RPT = NP // NS
EPT = EPG // NS
ECH = EPT // 128
ROWS_W = (B * NP) // NW
RCH = ROWS_W // 128

PPAD = 5104
CPAD = 408
OPAD = 5120


def _leaky(x, slope):
    return jnp.maximum(x, slope * x)


def _tc_proj(poi_pad, cat_pad, Wp, Wc):
    def body(poi_ref, cat_ref, wp_ref, wc_ref, op_ref, oc_ref):
        op_ref[0] = jnp.dot(poi_ref[...], wp_ref[...],
                            preferred_element_type=jnp.float32)
        oc_ref[0] = jnp.dot(cat_ref[...], wc_ref[...],
                            preferred_element_type=jnp.float32)

    return pl.pallas_call(
        body,
        grid=(NW,),
        in_specs=[
            pl.BlockSpec((PPAD, 304), lambda w: (0, 0)),
            pl.BlockSpec((CPAD, 112), lambda w: (0, 0)),
            pl.BlockSpec((304, CH), lambda w: (0, 0)),
            pl.BlockSpec((112, CH), lambda w: (0, 0)),
        ],
        out_specs=(pl.BlockSpec((1, PPAD, CH), lambda w: (w, 0, 0)),
                   pl.BlockSpec((1, CPAD, CH), lambda w: (w, 0, 0))),
        out_shape=(jax.ShapeDtypeStruct((NW, PPAD, CH), jnp.float32),
                   jax.ShapeDtypeStruct((NW, CPAD, CH), jnp.float32)),
    )(poi_pad, cat_pad, Wp, Wc)


def _sc_gather(poi_proj, cat_proj, pidx, cidx):
    mesh = plsc.VectorSubcoreMesh(core_axis_name="c", subcore_axis_name="s")

    @functools.partial(
        pl.kernel,
        out_type=jax.ShapeDtypeStruct((B * NP, CH), jnp.float32),
        mesh=mesh,
        scratch_types=[
            pltpu.VMEM((RCH, 128), jnp.int32),
            pltpu.VMEM((RCH, 128), jnp.int32),
            pltpu.VMEM((128, CH), jnp.float32),
            pltpu.VMEM((128, CH), jnp.float32),
            pltpu.SemaphoreType.DMA,
            pltpu.SemaphoreType.DMA,
            pltpu.SemaphoreType.DMA,
        ],
    )
    def k(poi_hbm, cat_hbm, pidx_hbm, cidx_hbm, xin_hbm,
          pib, cib, prow, crow, sem0, sem1, semw):
        wid = lax.axis_index("c") * NS + lax.axis_index("s")
        pltpu.sync_copy(pidx_hbm.at[wid], pib)
        pltpu.sync_copy(cidx_hbm.at[wid], cib)

        def chunk(ch, _):
            base = wid * ROWS_W + ch * 128
            cp = pltpu.async_copy(poi_hbm.at[pib.at[ch]], prow, sem0)
            cc = pltpu.async_copy(cat_hbm.at[cib.at[ch]], crow, sem1)
            cp.wait()
            cc.wait()

            def row(r, _):
                for kk in range(8):
                    sl = pl.ds(kk * 16, 16)
                    prow[r, sl] = prow[r, sl] + crow[r, sl]
                return ()
            lax.fori_loop(0, 128, row, ())
            pltpu.async_copy(prow, xin_hbm.at[pl.ds(base, 128)], semw).wait()
            return ()
        lax.fori_loop(0, RCH, chunk, ())

    return k(poi_proj, cat_proj, pidx, cidx)


def _sc_build_adj(gidx, ew, ng):
    mesh = plsc.VectorSubcoreMesh(core_axis_name="c", subcore_axis_name="s")
    GPC = ng // NC

    @functools.partial(
        pl.kernel,
        out_type=(jax.ShapeDtypeStruct((ng, NP, NP), jnp.float32),
                  jax.ShapeDtypeStruct((ng, NP, NP), jnp.float32)),
        mesh=mesh,
        scratch_types=[
            pltpu.VMEM((ECH, 128), jnp.int32),
            pltpu.VMEM((ECH, 128), jnp.float32),
            pltpu.VMEM((128,), jnp.float32),
            pltpu.VMEM((TPS,), jnp.float32),
            pltpu.VMEM_SHARED((NPNP,), jnp.float32),
            pltpu.VMEM_SHARED((NPNP,), jnp.float32),
            pltpu.SemaphoreType.DMA,
            pltpu.SemaphoreType.DMA,
            pltpu.SemaphoreType.DMA,
        ],
    )
    def k(gidx_hbm, ew_hbm, outc_hbm, outw_hbm,
          ib, wb, ones, zb, accc, accw, sems, semd, seme):
        c = lax.axis_index("c")
        s = lax.axis_index("s")

        def fill16(ref, val, n):
            def bodyf(i, _):
                ref[pl.ds(i * 16, 16)] = jnp.full((16,), val, jnp.float32)
                return ()
            lax.fori_loop(0, n, bodyf, ())
        fill16(ones, 1.0, 8)
        fill16(zb, 0.0, TPS // 16)

        sl = pl.ds(s * TPS, TPS)
        pltpu.async_copy(zb, accc.at[sl], semd)
        pltpu.async_copy(zb, accw.at[sl], semd)
        cpe0 = pltpu.async_copy(gidx_hbm.at[c * GPC, s], ib, seme)
        cpe1 = pltpu.async_copy(ew_hbm.at[c * GPC, s], wb, seme)
        pltpu.make_async_copy(zb, accc.at[sl], semd).wait()
        pltpu.make_async_copy(zb, accw.at[sl], semd).wait()
        cpe0.wait()
        cpe1.wait()
        plsc.subcore_barrier()

        def per_graph(j, _):
            g = c * GPC + j
            with jax.named_scope("adj_scatter"):
                for i in range(ECH):
                    pltpu.async_copy(ones, accc.at[ib.at[i]], sems, add=True)
                    pltpu.async_copy(wb.at[i], accw.at[ib.at[i]], sems,
                                     add=True)
                for i in range(ECH):
                    pltpu.make_async_copy(ones, accc.at[ib.at[i]],
                                          sems).wait()
                    pltpu.make_async_copy(wb.at[i], accw.at[ib.at[i]],
                                          sems).wait()
            plsc.subcore_barrier()
            with jax.named_scope("adj_drain"):
                def drain_row(r, _):
                    rr = s * RPT + r
                    pltpu.async_copy(accc.at[pl.ds(rr * NP, NP)],
                                     outc_hbm.at[g, rr, :], semd)
                    pltpu.async_copy(accw.at[pl.ds(rr * NP, NP)],
                                     outw_hbm.at[g, rr, :], semd)
                    return ()
                lax.fori_loop(0, RPT, drain_row, ())

                def wait_row(r, _):
                    pltpu.make_async_copy(accc.at[pl.ds(0, NP)],
                                          outc_hbm.at[g, 0, :], semd).wait()
                    pltpu.make_async_copy(accw.at[pl.ds(0, NP)],
                                          outw_hbm.at[g, 0, :], semd).wait()
                    return ()
                lax.fori_loop(0, RPT, wait_row, ())

            @pl.when(j + 1 < GPC)
            def _():
                with jax.named_scope("adj_refill"):
                    ce0 = pltpu.async_copy(gidx_hbm.at[g + 1, s], ib, seme)
                    ce1 = pltpu.async_copy(ew_hbm.at[g + 1, s], wb, seme)
                    zc = pltpu.async_copy(zb, accc.at[sl], semd)
                    zw = pltpu.async_copy(zb, accw.at[sl], semd)
                    zc.wait()
                    zw.wait()
                    ce0.wait()
                    ce1.wait()
            plsc.subcore_barrier()
            return ()
        lax.fori_loop(0, GPC, per_graph, ())

    return k(gidx, ew)


def _tc_gnn(xine, f3, cnt, wsum, W3p, bin2, Wgx, BG,
            Woutp, bo, ng):
    def body(xp_ref, f3_ref, cnt_ref, ws_ref, w3_ref, bin_ref,
             wg_ref, bg_ref, wo_ref, bo_ref, out_ref):
        cntm = cnt_ref[0]
        wsm = ws_ref[0]
        xin = (xp_ref[0]
               + jnp.dot(f3_ref[0], w3_ref[...],
                         preferred_element_type=jnp.float32))
        deg = jnp.sum(wsm, axis=1, keepdims=True) + 1.0
        dis = 1.0 / jnp.sqrt(deg)
        dis2 = dis * dis

        def gcn_apply(h):
            t = jnp.dot(wsm, h * dis, preferred_element_type=jnp.float32)
            return dis * t + dis2 * h

        f = _leaky(gcn_apply(xin) + bin_ref[0][None, :], 0.01)

        r = lax.broadcasted_iota(jnp.int32, (NP, NP), 0)
        cidx = lax.broadcasted_iota(jnp.int32, (NP, NP), 1)
        cpi = cntm + jnp.where(r == cidx, 1.0, 0.0)
        lane136 = lax.broadcasted_iota(jnp.int32, (NP, 136), 1)

        def gat(x, i):
            hx = jnp.dot(x, wg_ref[i], preferred_element_type=jnp.float32)
            asrc = hx[:, 128]
            adst = hx[:, 129]
            e = adst[:, None] + asrc[None, :]
            e = jnp.maximum(e, 0.2 * e)
            m0 = adst + jnp.max(asrc)
            m = jnp.maximum(m0, 0.2 * m0)[:, None]
            p = cpi * jnp.exp(e - m)
            hext = jnp.where(lane136 == 130, 1.0, hx)
            agg2 = jnp.dot(p, hext, preferred_element_type=jnp.float32)
            den = agg2[:, 130:131]
            return agg2[:, :CH] / den + bg_ref[i][None, :]

        for i in range(3):
            t = gat(f, i)
            y = _leaky(t, 0.01) + t
            y2 = gat(y, i)
            f = _leaky(y2, 0.01)

        h2 = jnp.dot(f, wo_ref[...], preferred_element_type=jnp.float32)
        f2 = _leaky(gcn_apply(h2) + bo_ref[0][None, :], 0.01)
        out_ref[0, 0] = f2[:, 0]

    grid = (ng,)
    return pl.pallas_call(
        body,
        grid=grid,
        in_specs=[
            pl.BlockSpec((1, NP, CH), lambda g: (g, 0, 0)),
            pl.BlockSpec((1, NP, 8), lambda g: (g, 0, 0)),
            pl.BlockSpec((1, NP, NP), lambda g: (g, 0, 0)),
            pl.BlockSpec((1, NP, NP), lambda g: (g, 0, 0)),
            pl.BlockSpec((8, CH), lambda g: (0, 0)),
            pl.BlockSpec((1, CH), lambda g: (0, 0)),
            pl.BlockSpec((3, CH, 136), lambda g: (0, 0, 0)),
            pl.BlockSpec((8, CH), lambda g: (0, 0)),
            pl.BlockSpec((CH, 8), lambda g: (0, 0)),
            pl.BlockSpec((1, 8), lambda g: (0, 0)),
        ],
        out_specs=pl.BlockSpec((1, 1, NP), lambda g: (g, 0, 0)),
        out_shape=jax.ShapeDtypeStruct((ng, 1, NP), jnp.float32),
    )(xine, f3, cnt, wsum, W3p, bin2, Wgx, BG, Woutp, bo)


def _tc_head(fgr, W1p, b1, W2p, b2p):
    def body(f_ref, w1_ref, b1_ref, w2_ref, b2_ref, out_ref):
        h = jnp.maximum(
            jnp.dot(f_ref[...], w1_ref[...],
                    preferred_element_type=jnp.float32) + b1_ref[...], 0.0)
        out_ref[...] = jnp.maximum(
            jnp.dot(h, w2_ref[...],
                    preferred_element_type=jnp.float32) + b2_ref[...], 0.0)

    return pl.pallas_call(
        body,
        out_shape=jax.ShapeDtypeStruct((B, OPAD), jnp.float32),
    )(fgr, W1p, b1, W2p, b2p)


def kernel(feature, edges, weight, params):
    f32 = jnp.float32

    poi_idx = feature[:, :, 0].astype(jnp.int32)
    cat_idx = feature[:, :, 1].astype(jnp.int32)

    nar = jnp.arange(NP, dtype=jnp.int32)[None, :]
    ppad = PLEN + nar % (PPAD - PLEN)
    cpad = CLEN + nar % (CPAD - CLEN)
    pidx = jnp.broadcast_to(ppad, (B, NP)).at[:, :NODE].set(poi_idx)
    cidx = jnp.broadcast_to(cpad, (B, NP)).at[:, :NODE].set(cat_idx)
    woff = jnp.arange(NW, dtype=jnp.int32)[:, None, None]
    pidx = pidx.reshape(NW, RCH, 128) + woff * PPAD
    cidx = cidx.reshape(NW, RCH, 128) + woff * CPAD

    src = edges[:, 0, :]
    dst = edges[:, 1, :]
    gidx = (dst * NP + src).reshape(B, NS, ECH, 128)
    ew = weight[:, :, 1].reshape(B, NS, ECH, 128).astype(f32)

    W_in = params['W_in']
    poi_pad = jnp.zeros((PPAD, 304), f32).at[:PLEN, :PDIM].set(
        params['poi_table'])
    Wp = jnp.zeros((304, CH), f32).at[:PDIM].set(W_in[:PDIM])
    cat_pad = jnp.zeros((CPAD, 112), f32).at[:CLEN, :CDIM].set(
        params['cat_table'])
    Wc = jnp.zeros((112, CH), f32).at[:CDIM].set(W_in[PDIM:PDIM + CDIM])

    f3 = jnp.zeros((B, NP, 8), f32).at[:, :NODE, 0:3].set(feature[:, :, 2:5])
    W3p = jnp.zeros((8, CH), f32).at[0:3].set(W_in[PDIM + CDIM:])
    bin2 = params['b_in'][None, :]

    Wg_all = jnp.stack([params['Wg%d' % i] for i in range(3)])
    Wgx = jnp.zeros((3, CH, 136), f32)
    Wgx = Wgx.at[:, :, :CH].set(Wg_all)
    Wgx = Wgx.at[:, :, CH].set(jnp.einsum(
        'ikc,ic->ik', Wg_all, jnp.stack(
            [params['as%d' % i] for i in range(3)])))
    Wgx = Wgx.at[:, :, CH + 1].set(jnp.einsum(
        'ikc,ic->ik', Wg_all, jnp.stack(
            [params['ad%d' % i] for i in range(3)])))
    BG = jnp.zeros((8, CH), f32).at[0:3].set(
        jnp.stack([params['bg%d' % i] for i in range(3)]))

    Woutp = jnp.zeros((CH, 8), f32).at[:, 0].set(params['W_out'][:, 0])
    bo = jnp.broadcast_to(params['b_out'][0], (1, 8))

    W1p = jnp.zeros((NP, CH), f32).at[:NODE].set(params['W1'])
    b1 = params['b1'][None, :]
    W2p = jnp.zeros((CH, OPAD), f32).at[:, :PLEN].set(params['W2'])
    b2p = jnp.zeros((1, OPAD), f32).at[0, :PLEN].set(params['b2'])

    poi_proj, cat_proj = _tc_proj(poi_pad, cat_pad, Wp, Wc)
    xine = _sc_gather(poi_proj.reshape(NW * PPAD, CH),
                      cat_proj.reshape(NW * CPAD, CH),
                      pidx, cidx).reshape(B, NP, CH)

    NCHUNK = 4
    GC = B // NCHUNK
    fgs = []
    for k in range(NCHUNK):
        sl = slice(k * GC, (k + 1) * GC)
        cnt_k, wsum_k = _sc_build_adj(gidx[sl], ew[sl], GC)
        fgs.append(_tc_gnn(xine[sl], f3[sl], cnt_k, wsum_k, W3p, bin2,
                           Wgx, BG, Woutp, bo, GC))
    fgr = jnp.concatenate(fgs, axis=0).reshape(B, NP)
    out = _tc_head(fgr, W1p, b1, W2p, b2p)
    return out[:, :PLEN]
